# Initial kernel scaffold; baseline (speedup 1.0000x reference)
import jax
import jax.numpy as jnp
from jax.experimental import pallas as pl


def kernel(x, edge_index_x, ptr_x, y, edge_index_y, ptr_y, params):
    raise NotImplementedError("write your pallas kernel here")


# speedup vs baseline: 18.3063x; 18.3063x over previous
import functools

import numpy as np
import jax
import jax.numpy as jnp
from jax import lax
from jax.experimental import pallas as pl
from jax.experimental.pallas import tpu as pltpu
from jax.experimental.pallas import tpu_sc as plsc

D = 128
N = 10000
E = 320000
B = 16
NC = 2
NS = 16
W = 80
WE = 125
EPT = E // NS
NWIN = EPT // WE
ER = E // WE
RPT = N // NS
NPAD = 10240
LOG2 = float(np.log(2.0))

def _fill_zero_rows(zbuf):
  def row(r, _):
    for k in range(8):
      zbuf[r, pl.ds(k * 16, 16)] = jnp.zeros((16,), jnp.float32)
    return 0
  lax.fori_loop(0, 128, row, 0)


def _embdeg_body(table, ids2d, dst2d, embflat, deg, idsv, rows_v, dstv,
                 ones_v, zrow, deg_sh):
  c = lax.axis_index("c")
  s = lax.axis_index("s")

  def fill16(i, _):
    zrow[pl.ds(i * 16, 16)] = jnp.zeros((16,), jnp.float32)
    return 0
  lax.fori_loop(0, 40, fill16, 0)
  for i in range(8):
    ones_v[pl.ds(i * 16, 16)] = jnp.ones((16,), jnp.float32)

  pltpu.sync_copy(zrow, deg_sh.at[pl.ds(s * 640, 640)])
  plsc.subcore_barrier()

  pltpu.sync_copy(ids2d.at[pl.ds(c * 128 + s * 8, 8)], idsv)

  def gather_win(j, _):
    w = s * 8 + j
    @pl.when(w < 125)
    def _():
      pltpu.sync_copy(table.at[idsv.at[j]], rows_v)
      pltpu.sync_copy(rows_v, embflat.at[pl.ds(c * N + w * W, W)])
    return 0
  lax.fori_loop(0, 8, gather_win, 0)

  pltpu.sync_copy(dst2d.at[pl.ds(c * ER + s * NWIN, NWIN)], dstv)

  def hist(j, _):
    pltpu.sync_copy(ones_v.at[pl.ds(0, WE)], deg_sh.at[dstv.at[j]], add=True)
    return 0
  lax.fori_loop(0, NWIN, hist, 0)

  plsc.subcore_barrier()
  pltpu.sync_copy(deg_sh.at[pl.ds(s * 640, 640)], deg.at[c, pl.ds(s * 640, 640)])


def _segsum_body(zflat, src2d, dst2d, accflat, srcv, dstv, rows_v, acc_sh):
  c = lax.axis_index("c")
  s = lax.axis_index("s")

  _fill_zero_rows(rows_v)
  base = s * 640
  for q in range(5):
    pltpu.sync_copy(rows_v, acc_sh.at[pl.ds(base + q * 128, 128)])
  plsc.subcore_barrier()

  erow = c * ER + s * NWIN
  for half in range(2):
    pltpu.sync_copy(src2d.at[pl.ds(erow + half * (NWIN // 2), NWIN // 2)],
                    srcv)
    pltpu.sync_copy(dst2d.at[pl.ds(erow + half * (NWIN // 2), NWIN // 2)],
                    dstv)

    def win(j, _):
      pltpu.sync_copy(zflat.at[srcv.at[j]], rows_v.at[pl.ds(0, WE)])
      pltpu.sync_copy(rows_v.at[pl.ds(0, WE)], acc_sh.at[dstv.at[j]],
                      add=True)
      return 0
    lax.fori_loop(0, NWIN // 2, win, 0)

  plsc.subcore_barrier()
  pltpu.sync_copy(acc_sh.at[pl.ds(base, 640)],
                  accflat.at[pl.ds(c * NPAD + base, 640)])


@functools.lru_cache(maxsize=None)
def _sc_kernels():
  mesh = plsc.VectorSubcoreMesh(core_axis_name="c", subcore_axis_name="s",
                                num_cores=NC, num_subcores=NS)
  embdeg = functools.partial(
      pl.kernel,
      out_type=(
          jax.ShapeDtypeStruct((2 * N, D), jnp.float32),
          jax.ShapeDtypeStruct((2, NPAD), jnp.float32),
      ),
      mesh=mesh,
      scratch_types=(
          pltpu.VMEM((8, W), jnp.int32),
          pltpu.VMEM((W, D), jnp.float32),
          pltpu.VMEM((NWIN, WE), jnp.int32),
          pltpu.VMEM((128,), jnp.float32),
          pltpu.VMEM((640,), jnp.float32),
          pltpu.VMEM_SHARED((NPAD,), jnp.float32),
      ),
  )(_embdeg_body)
  segsum = functools.partial(
      pl.kernel,
      out_type=jax.ShapeDtypeStruct((2 * NPAD, D), jnp.float32),
      mesh=mesh,
      scratch_types=(
          pltpu.VMEM((NWIN // 2, WE), jnp.int32),
          pltpu.VMEM((NWIN // 2, WE), jnp.int32),
          pltpu.VMEM((128, D), jnp.float32),
          pltpu.VMEM_SHARED((NPAD, D), jnp.float32),
      ),
  )(_segsum_body)
  return embdeg, segsum


def _ln(x, g, b):
  m = jnp.mean(x, axis=-1, keepdims=True)
  v = jnp.mean((x - m) ** 2, axis=-1, keepdims=True)
  return (x - m) * lax.rsqrt(v + 1e-5) * g + b


def _matmul_t(x, w):
  return lax.dot_general(x, w, (((1,), (1,)), ((), ())),
                         preferred_element_type=jnp.float32)


def _softplus(x):
  return jnp.maximum(x, 0.0) + jnp.log1p(jnp.exp(-jnp.abs(x)))


def _t1_body(emb, w1, deg, z1, dinv):
  dv = lax.rsqrt(deg[0] + 1.0)
  z = _matmul_t(emb[0], w1[0]) * dv
  z1[...] = z[None]
  dinv[...] = dv[None]


def _t2_body(emb, z1, acc1, dinv, ws, bs, b1, g1, be1, w2, z2):
  dv = dinv[0]
  pre = dv * (acc1[0] + z1[0]) + b1[0]
  h = jax.nn.relu(_ln(pre, g1[0], be1[0]))
  u = _matmul_t(emb[0], ws[0]) + bs[0] + h
  z2[...] = (_matmul_t(u, w2[0]) * dv)[None]


def _t3_body(acc2, z2, dinv, b2, g2, be2, lw, lb, lg, lbe, lal, sw, sb,
             lp, gsum):
  dv = dinv[0]
  local = jax.nn.relu(_ln(dv * (acc2[0] + z2[0]) + b2[0], g2[0], be2[0]))
  rows = lax.broadcasted_iota(jnp.int32, (8, 5000), 1) // 625
  gid = lax.broadcasted_iota(jnp.int32, (8, 5000), 0)
  onehot = (rows == gid).astype(jnp.float32)
  gsum[...] = lax.dot_general(onehot, local, (((1,), (0,)), ((), ())),
                              preferred_element_type=jnp.float32)[None]
  h = local
  for i in range(3):
    a = lal[i]
    t = _ln(_matmul_t(h, lw[i]) + lb[i], lg[i], lbe[i])
    h = jnp.where(t >= 0, t, a * t)
  lp[...] = (h + _matmul_t(local, sw[...]) + sb[...])[None]


def _t4_body(lp, gsum, mw, mb, mg, mbe, mal, msw, msb, out):
  hblk = pl.program_id(0)

  def mlp2(xg):
    h = xg
    for i in range(3):
      t = _ln(_matmul_t(h, mw[i]) + mb[i], mg[i], mbe[i])
      h = jnp.where(t >= 0, t, mal[i] * t)
    return h + _matmul_t(xg, msw[...]) + msb[...]

  g1p = mlp2(gsum[0] / 625.0)
  g2p = mlp2(gsum[1] / 625.0)

  def contrib(l_blk, gp):
    res = _matmul_t(l_blk, gp)
    rowg = lax.broadcasted_iota(jnp.int32, (5000, B), 0) // 625 + hblk * 8
    colg = lax.broadcasted_iota(jnp.int32, (5000, B), 1)
    pos = (colg == rowg).astype(jnp.float32)
    spn = _softplus(-res)
    e_pos = jnp.sum(pos * (LOG2 - spn))
    e_neg = jnp.sum((1.0 - pos) * (spn + res - LOG2))
    return e_neg / (N * (B - 1)) - e_pos / N

  c = contrib(lp[0], g2p) + contrib(lp[1], g1p)

  cb = jnp.reshape(c, (1, 1))

  @pl.when(hblk == 0)
  def _():
    out[...] = cb

  @pl.when(hblk != 0)
  def _():
    out[...] = out[...] + cb


def _spec(shape, index_map):
  return pl.BlockSpec(shape, index_map)


def _t1_call(emb3, w1s, degv):
  return pl.pallas_call(
      _t1_body,
      grid=(2, 10),
      in_specs=[
          _spec((1, 1000, D), lambda v, i: (v, i, 0)),
          _spec((1, D, D), lambda v, i: (v, 0, 0)),
          _spec((1, 1000, 1), lambda v, i: (v, i, 0)),
      ],
      out_specs=[
          _spec((1, 1000, D), lambda v, i: (v, i, 0)),
          _spec((1, 1000, 1), lambda v, i: (v, i, 0)),
      ],
      out_shape=[
          jax.ShapeDtypeStruct((2, N, D), jnp.float32),
          jax.ShapeDtypeStruct((2, N, 1), jnp.float32),
      ],
  )(emb3, w1s, degv)


def _t2_call(emb3, z1, acc1, dinv, wss, bss, b1s, g1s, be1s, w2s):
  row = lambda v, i: (v, i, 0)
  par = lambda v, i: (v, 0, 0)
  return pl.pallas_call(
      _t2_body,
      grid=(2, 10),
      in_specs=[
          _spec((1, 1000, D), row), _spec((1, 1000, D), row),
          _spec((1, 1000, D), row), _spec((1, 1000, 1), row),
          _spec((1, D, D), par), _spec((1, 1, D), par), _spec((1, 1, D), par),
          _spec((1, 1, D), par), _spec((1, 1, D), par), _spec((1, D, D), par),
      ],
      out_specs=_spec((1, 1000, D), row),
      out_shape=jax.ShapeDtypeStruct((2, N, D), jnp.float32),
  )(emb3, z1, acc1, dinv, wss, bss, b1s, g1s, be1s, w2s)


def _t3_call(acc2, z2, dinv, b2s, g2s, be2s, lw, lb, lg, lbe, lal, sw, sb):
  row = lambda v, h: (v, h, 0)
  par = lambda v, h: (v, 0, 0)
  full3 = lambda v, h: (0, 0, 0)
  full2 = lambda v, h: (0, 0)
  return pl.pallas_call(
      _t3_body,
      grid=(2, 2),
      in_specs=[
          _spec((1, 5000, D), row), _spec((1, 5000, D), row),
          _spec((1, 5000, 1), row),
          _spec((1, 1, D), par), _spec((1, 1, D), par), _spec((1, 1, D), par),
          _spec((3, D, D), full3), _spec((3, 1, D), full3),
          _spec((3, 1, D), full3), _spec((3, 1, D), full3),
          _spec((3, 1, 1), full3),
          _spec((D, D), full2), _spec((1, D), full2),
      ],
      out_specs=[
          _spec((1, 5000, D), row),
          _spec((1, 8, D), row),
      ],
      out_shape=[
          jax.ShapeDtypeStruct((2, N, D), jnp.float32),
          jax.ShapeDtypeStruct((2, B, D), jnp.float32),
      ],
  )(acc2, z2, dinv, b2s, g2s, be2s, lw, lb, lg, lbe, lal, sw, sb)


def _t4_call(lp, gsum, mw, mb, mg, mbe, mal, msw, msb):
  full3 = lambda h: (0, 0, 0)
  full2 = lambda h: (0, 0)
  return pl.pallas_call(
      _t4_body,
      grid=(2,),
      in_specs=[
          _spec((2, 5000, D), lambda h: (0, h, 0)),
          _spec((2, B, D), full3),
          _spec((3, D, D), full3), _spec((3, 1, D), full3),
          _spec((3, 1, D), full3), _spec((3, 1, D), full3),
          _spec((3, 1, 1), full3),
          _spec((D, D), full2), _spec((1, D), full2),
      ],
      out_specs=_spec((1, 1), lambda h: (0, 0)),
      out_shape=jax.ShapeDtypeStruct((1, 1), jnp.float32),
  )(lp, gsum, mw, mb, mg, mbe, mal, msw, msb)


def _stack_enc(p, key, idx=None, sub=None):
  def leaf(enc):
    v = enc[key]
    if idx is not None:
      v = v[idx]
    if sub is not None:
      v = v[sub]
    return v
  return jnp.stack([leaf(p['enc1']), leaf(p['enc2'])])


def kernel(x, edge_index_x, ptr_x, y, edge_index_y, ptr_y, params):
  p = params
  table = p['emb']

  xi = x.astype(jnp.int32)
  yi = y.astype(jnp.int32)
  padi = jnp.zeros((NPAD - N,), jnp.int32)
  ids2d = jnp.concatenate([xi, padi, yi, padi]).reshape(2 * 128, W)

  sx, dx = edge_index_x[0], edge_index_x[1]
  sy, dy = edge_index_y[0], edge_index_y[1]
  src2d = jnp.concatenate([sx, sy + N]).reshape(2 * ER, WE)
  dst2d = jnp.concatenate([dx, dy]).reshape(2 * ER, WE)

  embdeg_k, segsum_k = _sc_kernels()
  embflat, deg2 = embdeg_k(table, ids2d, dst2d)
  emb3 = embflat.reshape(2, N, D)
  degv = deg2[:, :N].reshape(2, N, 1)

  w1s = _stack_enc(p, 'conv', idx=0, sub='W')
  z1, dinv = _t1_call(emb3, w1s, degv)

  acc1 = segsum_k(z1.reshape(2 * N, D), src2d,
                  dst2d).reshape(2, NPAD, D)[:, :N]

  wss = _stack_enc(p, 'fc_skip_W')
  bss = _stack_enc(p, 'fc_skip_b').reshape(2, 1, D)
  b1s = _stack_enc(p, 'conv', idx=0, sub='b').reshape(2, 1, D)
  g1s = _stack_enc(p, 'ln', idx=0, sub='g').reshape(2, 1, D)
  be1s = _stack_enc(p, 'ln', idx=0, sub='b').reshape(2, 1, D)
  w2s = _stack_enc(p, 'conv', idx=1, sub='W')
  z2 = _t2_call(emb3, z1, acc1, dinv, wss, bss, b1s, g1s, be1s, w2s)

  acc2 = segsum_k(z2.reshape(2 * N, D), src2d,
                  dst2d).reshape(2, NPAD, D)[:, :N]

  b2s = _stack_enc(p, 'conv', idx=1, sub='b').reshape(2, 1, D)
  g2s = _stack_enc(p, 'ln', idx=1, sub='g').reshape(2, 1, D)
  be2s = _stack_enc(p, 'ln', idx=1, sub='b').reshape(2, 1, D)
  m1 = p['mlp1']
  lw = jnp.stack([m1['lin'][i]['W'] for i in range(3)])
  lb = jnp.stack([m1['lin'][i]['b'] for i in range(3)]).reshape(3, 1, D)
  lg = jnp.stack([m1['ln'][i]['g'] for i in range(3)]).reshape(3, 1, D)
  lbe = jnp.stack([m1['ln'][i]['b'] for i in range(3)]).reshape(3, 1, D)
  lal = jnp.stack([m1['prelu'][i] for i in range(3)]).reshape(3, 1, 1)
  sw = m1['short_W']
  sb = m1['short_b'].reshape(1, D)
  lp, gsum = _t3_call(acc2, z2, dinv, b2s, g2s, be2s, lw, lb, lg, lbe, lal,
                      sw, sb)

  m2 = p['mlp2']
  mw = jnp.stack([m2['lin'][i]['W'] for i in range(3)])
  mb = jnp.stack([m2['lin'][i]['b'] for i in range(3)]).reshape(3, 1, D)
  mg = jnp.stack([m2['ln'][i]['g'] for i in range(3)]).reshape(3, 1, D)
  mbe = jnp.stack([m2['ln'][i]['b'] for i in range(3)]).reshape(3, 1, D)
  mal = jnp.stack([m2['prelu'][i] for i in range(3)]).reshape(3, 1, 1)
  msw = m2['short_W']
  msb = m2['short_b'].reshape(1, D)
  loss = _t4_call(lp, gsum, mw, mb, mg, mbe, mal, msw, msb)
  return loss[0, 0]


# speedup vs baseline: 22.7188x; 1.2410x over previous
import functools

import numpy as np
import jax
import jax.numpy as jnp
from jax import lax
from jax.experimental import pallas as pl
from jax.experimental.pallas import tpu as pltpu
from jax.experimental.pallas import tpu_sc as plsc

D = 128
N = 10000
E = 320000
B = 16
NC = 2
NS = 16
W = 80
WE = 125
EPT = E // NS
NWIN = EPT // WE
ER = E // WE
RPT = N // NS
NPAD = 10240
LOG2 = float(np.log(2.0))

def _fill_zero_rows(zbuf):
  def row(r, _):
    for k in range(8):
      zbuf[r, pl.ds(k * 16, 16)] = jnp.zeros((16,), jnp.float32)
    return 0
  lax.fori_loop(0, 128, row, 0)


def _embdeg_body(table, ids2d, dst2d, embflat, deg, idsv, rows_v, dstv,
                 ones_v, zrow, deg_sh):
  c = lax.axis_index("c")
  s = lax.axis_index("s")

  def fill16(i, _):
    zrow[pl.ds(i * 16, 16)] = jnp.zeros((16,), jnp.float32)
    return 0
  lax.fori_loop(0, 40, fill16, 0)
  for i in range(8):
    ones_v[pl.ds(i * 16, 16)] = jnp.ones((16,), jnp.float32)

  pltpu.sync_copy(zrow, deg_sh.at[pl.ds(s * 640, 640)])
  plsc.subcore_barrier()

  pltpu.sync_copy(ids2d.at[pl.ds(c * 128 + s * 8, 8)], idsv)

  def gather_win(j, _):
    w = s * 8 + j
    @pl.when(w < 125)
    def _():
      pltpu.sync_copy(table.at[idsv.at[j]], rows_v)
      pltpu.sync_copy(rows_v, embflat.at[pl.ds(c * N + w * W, W)])
    return 0
  lax.fori_loop(0, 8, gather_win, 0)

  pltpu.sync_copy(dst2d.at[pl.ds(c * ER + s * NWIN, NWIN)], dstv)

  def hist(j, _):
    pltpu.sync_copy(ones_v.at[pl.ds(0, WE)], deg_sh.at[dstv.at[j]], add=True)
    return 0
  lax.fori_loop(0, NWIN, hist, 0)

  plsc.subcore_barrier()
  pltpu.sync_copy(deg_sh.at[pl.ds(s * 640, 640)], deg.at[c, pl.ds(s * 640, 640)])


CH = 40


def _segsum_body(zflat, src2d, dst2d, accflat, srcv, dstv, buf0, buf1,
                 sg0, sg1, ss0, ss1, acc_sh):
  c = lax.axis_index("c")
  s = lax.axis_index("s")

  _fill_zero_rows(buf0)
  base = s * 640
  for q in range(5):
    pltpu.sync_copy(buf0, acc_sh.at[pl.ds(base + q * 128, 128)])
  plsc.subcore_barrier()

  erow = c * ER + s * NWIN
  b0 = buf0.at[pl.ds(0, WE)]
  b1 = buf1.at[pl.ds(0, WE)]

  for chunk in range(NWIN // CH):
    pltpu.sync_copy(src2d.at[pl.ds(erow + chunk * CH, CH)], srcv)
    pltpu.sync_copy(dst2d.at[pl.ds(erow + chunk * CH, CH)], dstv)

    def g_desc(j, b, sem):
      return pltpu.make_async_copy(zflat.at[srcv.at[j]], b, sem)

    def s_desc(j, b, sem):
      return pltpu.make_async_copy(b, acc_sh.at[dstv.at[j]], sem)

    g_desc(0, b0, sg0).start()

    def pair(k, _):
      w0 = 2 * k
      w1 = 2 * k + 1
      g_desc(w0, b0, sg0).wait()

      @pl.when(k > 0)
      def _():
        s_desc(w1 - 2, b1, ss1).wait()

      g_desc(w1, b1, sg1).start()
      s_desc(w0, b0, ss0).start(add=True)
      g_desc(w1, b1, sg1).wait()
      s_desc(w0, b0, ss0).wait()

      @pl.when(k < CH // 2 - 1)
      def _():
        g_desc(w0 + 2, b0, sg0).start()

      s_desc(w1, b1, ss1).start(add=True)
      return 0
    lax.fori_loop(0, CH // 2, pair, 0)
    s_desc(CH - 1, b1, ss1).wait()

  plsc.subcore_barrier()
  pltpu.sync_copy(acc_sh.at[pl.ds(base, 640)],
                  accflat.at[pl.ds(c * NPAD + base, 640)])


@functools.lru_cache(maxsize=None)
def _sc_kernels():
  mesh = plsc.VectorSubcoreMesh(core_axis_name="c", subcore_axis_name="s",
                                num_cores=NC, num_subcores=NS)
  embdeg = functools.partial(
      pl.kernel,
      out_type=(
          jax.ShapeDtypeStruct((2 * N, D), jnp.float32),
          jax.ShapeDtypeStruct((2, NPAD), jnp.float32),
      ),
      mesh=mesh,
      scratch_types=(
          pltpu.VMEM((8, W), jnp.int32),
          pltpu.VMEM((W, D), jnp.float32),
          pltpu.VMEM((NWIN, WE), jnp.int32),
          pltpu.VMEM((128,), jnp.float32),
          pltpu.VMEM((640,), jnp.float32),
          pltpu.VMEM_SHARED((NPAD,), jnp.float32),
      ),
  )(_embdeg_body)
  segsum = functools.partial(
      pl.kernel,
      out_type=jax.ShapeDtypeStruct((2 * NPAD, D), jnp.float32),
      mesh=mesh,
      scratch_types=(
          pltpu.VMEM((CH, WE), jnp.int32),
          pltpu.VMEM((CH, WE), jnp.int32),
          pltpu.VMEM((128, D), jnp.float32),
          pltpu.VMEM((128, D), jnp.float32),
          pltpu.SemaphoreType.DMA,
          pltpu.SemaphoreType.DMA,
          pltpu.SemaphoreType.DMA,
          pltpu.SemaphoreType.DMA,
          pltpu.VMEM_SHARED((NPAD, D), jnp.float32),
      ),
  )(_segsum_body)
  return embdeg, segsum


def _ln(x, g, b):
  m = jnp.mean(x, axis=-1, keepdims=True)
  v = jnp.mean((x - m) ** 2, axis=-1, keepdims=True)
  return (x - m) * lax.rsqrt(v + 1e-5) * g + b


def _matmul_t(x, w):
  return lax.dot_general(x, w, (((1,), (1,)), ((), ())),
                         preferred_element_type=jnp.float32)


def _softplus(x):
  return jnp.maximum(x, 0.0) + jnp.log1p(jnp.exp(-jnp.abs(x)))


def _t1_body(emb, w1, deg, z1, dinv):
  dv = lax.rsqrt(deg[0] + 1.0)
  z = _matmul_t(emb[0], w1[0]) * dv
  z1[...] = z[None]
  dinv[...] = dv[None]


def _t2_body(emb, z1, acc1, dinv, ws, bs, b1, g1, be1, w2, z2):
  dv = dinv[0]
  pre = dv * (acc1[0] + z1[0]) + b1[0]
  h = jax.nn.relu(_ln(pre, g1[0], be1[0]))
  u = _matmul_t(emb[0], ws[0]) + bs[0] + h
  z2[...] = (_matmul_t(u, w2[0]) * dv)[None]


def _t3_body(acc2, z2, dinv, b2, g2, be2, lw, lb, lg, lbe, lal, sw, sb,
             lp, gsum):
  dv = dinv[0]
  local = jax.nn.relu(_ln(dv * (acc2[0] + z2[0]) + b2[0], g2[0], be2[0]))
  rows = lax.broadcasted_iota(jnp.int32, (8, 5000), 1) // 625
  gid = lax.broadcasted_iota(jnp.int32, (8, 5000), 0)
  onehot = (rows == gid).astype(jnp.float32)
  gsum[...] = lax.dot_general(onehot, local, (((1,), (0,)), ((), ())),
                              preferred_element_type=jnp.float32)[None]
  h = local
  for i in range(3):
    a = lal[i]
    t = _ln(_matmul_t(h, lw[i]) + lb[i], lg[i], lbe[i])
    h = jnp.where(t >= 0, t, a * t)
  lp[...] = (h + _matmul_t(local, sw[...]) + sb[...])[None]


def _t4_body(lp, gsum, mw, mb, mg, mbe, mal, msw, msb, out):
  hblk = pl.program_id(0)

  def mlp2(xg):
    h = xg
    for i in range(3):
      t = _ln(_matmul_t(h, mw[i]) + mb[i], mg[i], mbe[i])
      h = jnp.where(t >= 0, t, mal[i] * t)
    return h + _matmul_t(xg, msw[...]) + msb[...]

  g1p = mlp2(gsum[0] / 625.0)
  g2p = mlp2(gsum[1] / 625.0)

  def contrib(l_blk, gp):
    res = _matmul_t(l_blk, gp)
    rowg = lax.broadcasted_iota(jnp.int32, (5000, B), 0) // 625 + hblk * 8
    colg = lax.broadcasted_iota(jnp.int32, (5000, B), 1)
    pos = (colg == rowg).astype(jnp.float32)
    spn = _softplus(-res)
    e_pos = jnp.sum(pos * (LOG2 - spn))
    e_neg = jnp.sum((1.0 - pos) * (spn + res - LOG2))
    return e_neg / (N * (B - 1)) - e_pos / N

  c = contrib(lp[0], g2p) + contrib(lp[1], g1p)

  cb = jnp.reshape(c, (1, 1))

  @pl.when(hblk == 0)
  def _():
    out[...] = cb

  @pl.when(hblk != 0)
  def _():
    out[...] = out[...] + cb


def _spec(shape, index_map):
  return pl.BlockSpec(shape, index_map)


def _t1_call(emb3, w1s, degv):
  return pl.pallas_call(
      _t1_body,
      grid=(2, 10),
      in_specs=[
          _spec((1, 1000, D), lambda v, i: (v, i, 0)),
          _spec((1, D, D), lambda v, i: (v, 0, 0)),
          _spec((1, 1000, 1), lambda v, i: (v, i, 0)),
      ],
      out_specs=[
          _spec((1, 1000, D), lambda v, i: (v, i, 0)),
          _spec((1, 1000, 1), lambda v, i: (v, i, 0)),
      ],
      out_shape=[
          jax.ShapeDtypeStruct((2, N, D), jnp.float32),
          jax.ShapeDtypeStruct((2, N, 1), jnp.float32),
      ],
  )(emb3, w1s, degv)


def _t2_call(emb3, z1, acc1, dinv, wss, bss, b1s, g1s, be1s, w2s):
  row = lambda v, i: (v, i, 0)
  par = lambda v, i: (v, 0, 0)
  return pl.pallas_call(
      _t2_body,
      grid=(2, 10),
      in_specs=[
          _spec((1, 1000, D), row), _spec((1, 1000, D), row),
          _spec((1, 1000, D), row), _spec((1, 1000, 1), row),
          _spec((1, D, D), par), _spec((1, 1, D), par), _spec((1, 1, D), par),
          _spec((1, 1, D), par), _spec((1, 1, D), par), _spec((1, D, D), par),
      ],
      out_specs=_spec((1, 1000, D), row),
      out_shape=jax.ShapeDtypeStruct((2, N, D), jnp.float32),
  )(emb3, z1, acc1, dinv, wss, bss, b1s, g1s, be1s, w2s)


def _t3_call(acc2, z2, dinv, b2s, g2s, be2s, lw, lb, lg, lbe, lal, sw, sb):
  row = lambda v, h: (v, h, 0)
  par = lambda v, h: (v, 0, 0)
  full3 = lambda v, h: (0, 0, 0)
  full2 = lambda v, h: (0, 0)
  return pl.pallas_call(
      _t3_body,
      grid=(2, 2),
      in_specs=[
          _spec((1, 5000, D), row), _spec((1, 5000, D), row),
          _spec((1, 5000, 1), row),
          _spec((1, 1, D), par), _spec((1, 1, D), par), _spec((1, 1, D), par),
          _spec((3, D, D), full3), _spec((3, 1, D), full3),
          _spec((3, 1, D), full3), _spec((3, 1, D), full3),
          _spec((3, 1, 1), full3),
          _spec((D, D), full2), _spec((1, D), full2),
      ],
      out_specs=[
          _spec((1, 5000, D), row),
          _spec((1, 8, D), row),
      ],
      out_shape=[
          jax.ShapeDtypeStruct((2, N, D), jnp.float32),
          jax.ShapeDtypeStruct((2, B, D), jnp.float32),
      ],
  )(acc2, z2, dinv, b2s, g2s, be2s, lw, lb, lg, lbe, lal, sw, sb)


def _t4_call(lp, gsum, mw, mb, mg, mbe, mal, msw, msb):
  full3 = lambda h: (0, 0, 0)
  full2 = lambda h: (0, 0)
  return pl.pallas_call(
      _t4_body,
      grid=(2,),
      in_specs=[
          _spec((2, 5000, D), lambda h: (0, h, 0)),
          _spec((2, B, D), full3),
          _spec((3, D, D), full3), _spec((3, 1, D), full3),
          _spec((3, 1, D), full3), _spec((3, 1, D), full3),
          _spec((3, 1, 1), full3),
          _spec((D, D), full2), _spec((1, D), full2),
      ],
      out_specs=_spec((1, 1), lambda h: (0, 0)),
      out_shape=jax.ShapeDtypeStruct((1, 1), jnp.float32),
  )(lp, gsum, mw, mb, mg, mbe, mal, msw, msb)


def _stack_enc(p, key, idx=None, sub=None):
  def leaf(enc):
    v = enc[key]
    if idx is not None:
      v = v[idx]
    if sub is not None:
      v = v[sub]
    return v
  return jnp.stack([leaf(p['enc1']), leaf(p['enc2'])])


def kernel(x, edge_index_x, ptr_x, y, edge_index_y, ptr_y, params):
  p = params
  table = p['emb']

  xi = x.astype(jnp.int32)
  yi = y.astype(jnp.int32)
  padi = jnp.zeros((NPAD - N,), jnp.int32)
  ids2d = jnp.concatenate([xi, padi, yi, padi]).reshape(2 * 128, W)

  sx, dx = edge_index_x[0], edge_index_x[1]
  sy, dy = edge_index_y[0], edge_index_y[1]
  src2d = jnp.concatenate([sx, sy + N]).reshape(2 * ER, WE)
  dst2d = jnp.concatenate([dx, dy]).reshape(2 * ER, WE)

  embdeg_k, segsum_k = _sc_kernels()
  embflat, deg2 = embdeg_k(table, ids2d, dst2d)
  emb3 = embflat.reshape(2, N, D)
  degv = deg2[:, :N].reshape(2, N, 1)

  w1s = _stack_enc(p, 'conv', idx=0, sub='W')
  z1, dinv = _t1_call(emb3, w1s, degv)

  acc1 = segsum_k(z1.reshape(2 * N, D), src2d,
                  dst2d).reshape(2, NPAD, D)[:, :N]

  wss = _stack_enc(p, 'fc_skip_W')
  bss = _stack_enc(p, 'fc_skip_b').reshape(2, 1, D)
  b1s = _stack_enc(p, 'conv', idx=0, sub='b').reshape(2, 1, D)
  g1s = _stack_enc(p, 'ln', idx=0, sub='g').reshape(2, 1, D)
  be1s = _stack_enc(p, 'ln', idx=0, sub='b').reshape(2, 1, D)
  w2s = _stack_enc(p, 'conv', idx=1, sub='W')
  z2 = _t2_call(emb3, z1, acc1, dinv, wss, bss, b1s, g1s, be1s, w2s)

  acc2 = segsum_k(z2.reshape(2 * N, D), src2d,
                  dst2d).reshape(2, NPAD, D)[:, :N]

  b2s = _stack_enc(p, 'conv', idx=1, sub='b').reshape(2, 1, D)
  g2s = _stack_enc(p, 'ln', idx=1, sub='g').reshape(2, 1, D)
  be2s = _stack_enc(p, 'ln', idx=1, sub='b').reshape(2, 1, D)
  m1 = p['mlp1']
  lw = jnp.stack([m1['lin'][i]['W'] for i in range(3)])
  lb = jnp.stack([m1['lin'][i]['b'] for i in range(3)]).reshape(3, 1, D)
  lg = jnp.stack([m1['ln'][i]['g'] for i in range(3)]).reshape(3, 1, D)
  lbe = jnp.stack([m1['ln'][i]['b'] for i in range(3)]).reshape(3, 1, D)
  lal = jnp.stack([m1['prelu'][i] for i in range(3)]).reshape(3, 1, 1)
  sw = m1['short_W']
  sb = m1['short_b'].reshape(1, D)
  lp, gsum = _t3_call(acc2, z2, dinv, b2s, g2s, be2s, lw, lb, lg, lbe, lal,
                      sw, sb)

  m2 = p['mlp2']
  mw = jnp.stack([m2['lin'][i]['W'] for i in range(3)])
  mb = jnp.stack([m2['lin'][i]['b'] for i in range(3)]).reshape(3, 1, D)
  mg = jnp.stack([m2['ln'][i]['g'] for i in range(3)]).reshape(3, 1, D)
  mbe = jnp.stack([m2['ln'][i]['b'] for i in range(3)]).reshape(3, 1, D)
  mal = jnp.stack([m2['prelu'][i] for i in range(3)]).reshape(3, 1, 1)
  msw = m2['short_W']
  msb = m2['short_b'].reshape(1, D)
  loss = _t4_call(lp, gsum, mw, mb, mg, mbe, mal, msw, msb)
  return loss[0, 0]


# speedup vs baseline: 23.8709x; 1.0507x over previous
import functools

import numpy as np
import jax
import jax.numpy as jnp
from jax import lax
from jax.experimental import pallas as pl
from jax.experimental.pallas import tpu as pltpu
from jax.experimental.pallas import tpu_sc as plsc

D = 128
N = 10000
E = 320000
B = 16
NC = 2
NS = 16
W = 80
WE = 125
EPT = E // NS
NWIN = EPT // WE
ER = E // WE
RPT = N // NS
NPAD = 10240
LOG2 = float(np.log(2.0))

def _fill_zero_rows(zbuf):
  def row(r, _):
    for k in range(8):
      zbuf[r, pl.ds(k * 16, 16)] = jnp.zeros((16,), jnp.float32)
    return 0
  lax.fori_loop(0, 128, row, 0)


def _embdeg_body(table, ids2d, dst2d, embflat, deg, idsv, rows_v, dstv,
                 ones_v, zrow, deg_sh):
  c = lax.axis_index("c")
  s = lax.axis_index("s")

  def fill16(i, _):
    zrow[pl.ds(i * 16, 16)] = jnp.zeros((16,), jnp.float32)
    return 0
  lax.fori_loop(0, 40, fill16, 0)
  for i in range(8):
    ones_v[pl.ds(i * 16, 16)] = jnp.ones((16,), jnp.float32)

  pltpu.sync_copy(zrow, deg_sh.at[pl.ds(s * 640, 640)])
  plsc.subcore_barrier()

  pltpu.sync_copy(ids2d.at[pl.ds(c * 128 + s * 8, 8)], idsv)

  def gather_win(j, _):
    w = s * 8 + j
    @pl.when(w < 125)
    def _():
      pltpu.sync_copy(table.at[idsv.at[j]], rows_v)
      pltpu.sync_copy(rows_v, embflat.at[pl.ds(c * N + w * W, W)])
    return 0
  lax.fori_loop(0, 8, gather_win, 0)

  pltpu.sync_copy(dst2d.at[pl.ds(c * ER + s * NWIN, NWIN)], dstv)

  def hist(j, _):
    pltpu.sync_copy(ones_v.at[pl.ds(0, WE)], deg_sh.at[dstv.at[j]], add=True)
    return 0
  lax.fori_loop(0, NWIN, hist, 0)

  plsc.subcore_barrier()
  pltpu.sync_copy(deg_sh.at[pl.ds(s * 640, 640)], deg.at[c, pl.ds(s * 640, 640)])


CH = 40


def _segsum_body(zflat, src2d, dst2d, accflat, srcv, dstv, buf0, buf1,
                 sg0, sg1, ss0, ss1, acc_sh, *, nwin_t, sc_rows):
  c = lax.axis_index("c")
  s = lax.axis_index("s")

  _fill_zero_rows(buf0)
  base = s * 640
  for q in range(5):
    pltpu.sync_copy(buf0, acc_sh.at[pl.ds(base + q * 128, 128)])
  plsc.subcore_barrier()

  erow = c * sc_rows + s * nwin_t
  b0 = buf0.at[pl.ds(0, WE)]
  b1 = buf1.at[pl.ds(0, WE)]

  for chunk in range(nwin_t // CH):
    pltpu.sync_copy(src2d.at[pl.ds(erow + chunk * CH, CH)], srcv)
    pltpu.sync_copy(dst2d.at[pl.ds(erow + chunk * CH, CH)], dstv)

    def g_desc(j, b, sem):
      return pltpu.make_async_copy(zflat.at[srcv.at[j]], b, sem)

    def s_desc(j, b, sem):
      return pltpu.make_async_copy(b, acc_sh.at[dstv.at[j]], sem)

    g_desc(0, b0, sg0).start()

    def pair(k, _):
      w0 = 2 * k
      w1 = 2 * k + 1
      g_desc(w0, b0, sg0).wait()

      @pl.when(k > 0)
      def _():
        s_desc(w1 - 2, b1, ss1).wait()

      g_desc(w1, b1, sg1).start()
      s_desc(w0, b0, ss0).start(add=True)
      g_desc(w1, b1, sg1).wait()
      s_desc(w0, b0, ss0).wait()

      @pl.when(k < CH // 2 - 1)
      def _():
        g_desc(w0 + 2, b0, sg0).start()

      s_desc(w1, b1, ss1).start(add=True)
      return 0
    lax.fori_loop(0, CH // 2, pair, 0)
    s_desc(CH - 1, b1, ss1).wait()

  plsc.subcore_barrier()
  pltpu.sync_copy(acc_sh.at[pl.ds(base, 640)],
                  accflat.at[pl.ds(c * NPAD + base, 640)])


@functools.lru_cache(maxsize=None)
def _sc_kernels():
  mesh = plsc.VectorSubcoreMesh(core_axis_name="c", subcore_axis_name="s",
                                num_cores=NC, num_subcores=NS)
  embdeg = functools.partial(
      pl.kernel,
      out_type=(
          jax.ShapeDtypeStruct((2 * N, D), jnp.float32),
          jax.ShapeDtypeStruct((2, NPAD), jnp.float32),
      ),
      mesh=mesh,
      scratch_types=(
          pltpu.VMEM((8, W), jnp.int32),
          pltpu.VMEM((W, D), jnp.float32),
          pltpu.VMEM((NWIN, WE), jnp.int32),
          pltpu.VMEM((128,), jnp.float32),
          pltpu.VMEM((640,), jnp.float32),
          pltpu.VMEM_SHARED((NPAD,), jnp.float32),
      ),
  )(_embdeg_body)
  nwin_t = E // WE // (NC * NS)
  body = functools.partial(_segsum_body, nwin_t=nwin_t, sc_rows=nwin_t * NS)
  segsum = functools.partial(
      pl.kernel,
      out_type=jax.ShapeDtypeStruct((2 * NPAD, D), jnp.float32),
      mesh=mesh,
      scratch_types=(
          pltpu.VMEM((CH, WE), jnp.int32),
          pltpu.VMEM((CH, WE), jnp.int32),
          pltpu.VMEM((128, D), jnp.float32),
          pltpu.VMEM((128, D), jnp.float32),
          pltpu.SemaphoreType.DMA,
          pltpu.SemaphoreType.DMA,
          pltpu.SemaphoreType.DMA,
          pltpu.SemaphoreType.DMA,
          pltpu.VMEM_SHARED((NPAD, D), jnp.float32),
      ),
  )(lambda *a: body(*a))
  return embdeg, segsum


def _ln(x, g, b):
  m = jnp.mean(x, axis=-1, keepdims=True)
  v = jnp.mean((x - m) ** 2, axis=-1, keepdims=True)
  return (x - m) * lax.rsqrt(v + 1e-5) * g + b


def _matmul_t(x, w):
  return lax.dot_general(x, w, (((1,), (1,)), ((), ())),
                         preferred_element_type=jnp.float32)


def _softplus(x):
  return jnp.maximum(x, 0.0) + jnp.log1p(jnp.exp(-jnp.abs(x)))


def _t1_body(emb, w1, deg, z1, dinv):
  dv = lax.rsqrt(deg[0] + 1.0)
  z = _matmul_t(emb[0], w1[0]) * dv
  z1[...] = z[None]
  dinv[...] = dv[None]


def _t2_body(emb, z1, acca, accb, dinv, ws, bs, b1, g1, be1, w2, z2):
  dv = dinv[0]
  pre = dv * (acca[0] + accb[0] + z1[0]) + b1[...]
  h = jax.nn.relu(_ln(pre, g1[...], be1[...]))
  u = _matmul_t(emb[0], ws[...]) + bs[...] + h
  z2[...] = _matmul_t(u, w2[...]) * dv


def _t3_body(acca, accb, z2, dinv, b2, g2, be2, lw, lb, lg, lbe, lal, sw, sb,
             lp, gsum):
  dv = dinv[0]
  local = jax.nn.relu(
      _ln(dv * (acca[0] + accb[0] + z2[...]) + b2[...], g2[...], be2[...]))
  rows = lax.broadcasted_iota(jnp.int32, (8, 5000), 1) // 625
  gid = lax.broadcasted_iota(jnp.int32, (8, 5000), 0)
  onehot = (rows == gid).astype(jnp.float32)
  gsum[...] = lax.dot_general(onehot, local, (((1,), (0,)), ((), ())),
                              preferred_element_type=jnp.float32)
  h = local
  for i in range(3):
    a = lal[i]
    t = _ln(_matmul_t(h, lw[i]) + lb[i], lg[i], lbe[i])
    h = jnp.where(t >= 0, t, a * t)
  lp[...] = h + _matmul_t(local, sw[...]) + sb[...]


def _t4_body(lp1, lp2, gsum1, gsum2, mw, mb, mg, mbe, mal, msw, msb, out):
  hblk = pl.program_id(0)

  def mlp2(xg):
    h = xg
    for i in range(3):
      t = _ln(_matmul_t(h, mw[i]) + mb[i], mg[i], mbe[i])
      h = jnp.where(t >= 0, t, mal[i] * t)
    return h + _matmul_t(xg, msw[...]) + msb[...]

  g1p = mlp2(gsum1[...] / 625.0)
  g2p = mlp2(gsum2[...] / 625.0)

  def contrib(l_blk, gp):
    res = _matmul_t(l_blk, gp)
    rowg = lax.broadcasted_iota(jnp.int32, (5000, B), 0) // 625 + hblk * 8
    colg = lax.broadcasted_iota(jnp.int32, (5000, B), 1)
    pos = (colg == rowg).astype(jnp.float32)
    spn = _softplus(-res)
    e_pos = jnp.sum(pos * (LOG2 - spn))
    e_neg = jnp.sum((1.0 - pos) * (spn + res - LOG2))
    return e_neg / (N * (B - 1)) - e_pos / N

  c = contrib(lp1[...], g2p) + contrib(lp2[...], g1p)

  cb = jnp.reshape(c, (1, 1))

  @pl.when(hblk == 0)
  def _():
    out[...] = cb

  @pl.when(hblk != 0)
  def _():
    out[...] = out[...] + cb


def _spec(shape, index_map):
  return pl.BlockSpec(shape, index_map)


def _t1_call(emb3, w1s, degv):
  return pl.pallas_call(
      _t1_body,
      grid=(2, 10),
      in_specs=[
          _spec((1, 1000, D), lambda v, i: (v, i, 0)),
          _spec((1, D, D), lambda v, i: (v, 0, 0)),
          _spec((1, 1000, 1), lambda v, i: (v, i, 0)),
      ],
      out_specs=[
          _spec((1, 1000, D), lambda v, i: (v, i, 0)),
          _spec((1, 1000, 1), lambda v, i: (v, i, 0)),
      ],
      out_shape=[
          jax.ShapeDtypeStruct((2, N, D), jnp.float32),
          jax.ShapeDtypeStruct((2, N, 1), jnp.float32),
      ],
  )(emb3, w1s, degv)


def _t2_call(v, emb3, z1, acc3, dinv, ws, bs, b1, g1, be1, w2):
  row = lambda i: (v, i, 0)
  full2 = lambda i: (0, 0)
  return pl.pallas_call(
      _t2_body,
      grid=(10,),
      in_specs=[
          _spec((1, 1000, D), row), _spec((1, 1000, D), row),
          _spec((1, 1000, D), lambda i: (0, i, 0)),
          _spec((1, 1000, D), lambda i: (1, i, 0)),
          _spec((1, 1000, 1), row),
          _spec((D, D), full2), _spec((1, D), full2), _spec((1, D), full2),
          _spec((1, D), full2), _spec((1, D), full2), _spec((D, D), full2),
      ],
      out_specs=_spec((1000, D), lambda i: (i, 0)),
      out_shape=jax.ShapeDtypeStruct((N, D), jnp.float32),
  )(emb3, z1, acc3, acc3, dinv, ws, bs, b1, g1, be1, w2)


def _t3_call(v, acc3, z2, dinv, b2, g2, be2, lw, lb, lg, lbe, lal, sw, sb):
  full3 = lambda h: (0, 0, 0)
  full2 = lambda h: (0, 0)
  return pl.pallas_call(
      _t3_body,
      grid=(2,),
      in_specs=[
          _spec((1, 5000, D), lambda h: (0, h, 0)),
          _spec((1, 5000, D), lambda h: (1, h, 0)),
          _spec((5000, D), lambda h: (h, 0)),
          _spec((1, 5000, 1), lambda h: (v, h, 0)),
          _spec((1, D), full2), _spec((1, D), full2), _spec((1, D), full2),
          _spec((3, D, D), full3), _spec((3, 1, D), full3),
          _spec((3, 1, D), full3), _spec((3, 1, D), full3),
          _spec((3, 1, 1), full3),
          _spec((D, D), full2), _spec((1, D), full2),
      ],
      out_specs=[
          _spec((5000, D), lambda h: (h, 0)),
          _spec((8, D), lambda h: (h, 0)),
      ],
      out_shape=[
          jax.ShapeDtypeStruct((N, D), jnp.float32),
          jax.ShapeDtypeStruct((B, D), jnp.float32),
      ],
  )(acc3, acc3, z2, dinv, b2, g2, be2, lw, lb, lg, lbe, lal, sw, sb)


def _t4_call(lp1, lp2, gsum1, gsum2, mw, mb, mg, mbe, mal, msw, msb):
  full3 = lambda h: (0, 0, 0)
  full2 = lambda h: (0, 0)
  row = lambda h: (h, 0)
  return pl.pallas_call(
      _t4_body,
      grid=(2,),
      in_specs=[
          _spec((5000, D), row), _spec((5000, D), row),
          _spec((B, D), full2), _spec((B, D), full2),
          _spec((3, D, D), full3), _spec((3, 1, D), full3),
          _spec((3, 1, D), full3), _spec((3, 1, D), full3),
          _spec((3, 1, 1), full3),
          _spec((D, D), full2), _spec((1, D), full2),
      ],
      out_specs=_spec((1, 1), lambda h: (0, 0)),
      out_shape=jax.ShapeDtypeStruct((1, 1), jnp.float32),
  )(lp1, lp2, gsum1, gsum2, mw, mb, mg, mbe, mal, msw, msb)


def _stack_enc(p, key, idx=None, sub=None):
  def leaf(enc):
    v = enc[key]
    if idx is not None:
      v = v[idx]
    if sub is not None:
      v = v[sub]
    return v
  return jnp.stack([leaf(p['enc1']), leaf(p['enc2'])])


def kernel(x, edge_index_x, ptr_x, y, edge_index_y, ptr_y, params):
  p = params
  table = p['emb']

  xi = x.astype(jnp.int32)
  yi = y.astype(jnp.int32)
  padi = jnp.zeros((NPAD - N,), jnp.int32)
  ids2d = jnp.concatenate([xi, padi, yi, padi]).reshape(2 * 128, W)

  sx, dx = edge_index_x[0], edge_index_x[1]
  sy, dy = edge_index_y[0], edge_index_y[1]
  srcx = sx.reshape(ER, WE)
  srcy_sh = (sy + N).reshape(ER, WE)
  srcy = sy.reshape(ER, WE)
  dstx = dx.reshape(ER, WE)
  dsty = dy.reshape(ER, WE)
  dst_both = jnp.concatenate([dx, dy]).reshape(2 * ER, WE)

  embdeg_k, segsum_k = _sc_kernels()
  embflat, deg2 = embdeg_k(table, ids2d, dst_both)
  emb3 = embflat.reshape(2, N, D)
  degv = deg2[:, :N].reshape(2, N, 1)

  w1s = _stack_enc(p, 'conv', idx=0, sub='W')
  z1, dinv = _t1_call(emb3, w1s, degv)
  z1flat = z1.reshape(2 * N, D)

  e1, e2 = p['enc1'], p['enc2']
  m1, m2 = p['mlp1'], p['mlp2']

  def enc2_params(e):
    return (e['fc_skip_W'], e['fc_skip_b'].reshape(1, D),
            e['conv'][0]['b'].reshape(1, D), e['ln'][0]['g'].reshape(1, D),
            e['ln'][0]['b'].reshape(1, D), e['conv'][1]['W'])

  def enc3_params(e):
    return (e['conv'][1]['b'].reshape(1, D), e['ln'][1]['g'].reshape(1, D),
            e['ln'][1]['b'].reshape(1, D))

  lw = jnp.stack([m1['lin'][i]['W'] for i in range(3)])
  lb = jnp.stack([m1['lin'][i]['b'] for i in range(3)]).reshape(3, 1, D)
  lg = jnp.stack([m1['ln'][i]['g'] for i in range(3)]).reshape(3, 1, D)
  lbe = jnp.stack([m1['ln'][i]['b'] for i in range(3)]).reshape(3, 1, D)
  lal = jnp.stack([m1['prelu'][i] for i in range(3)]).reshape(3, 1, 1)
  sw = m1['short_W']
  sb = m1['short_b'].reshape(1, D)
  mlp1p = (lw, lb, lg, lbe, lal, sw, sb)

  acc1x = segsum_k(z1flat, srcx, dstx).reshape(2, NPAD, D)
  acc1y = segsum_k(z1flat, srcy_sh, dsty).reshape(2, NPAD, D)
  z2x = _t2_call(0, emb3, z1, acc1x, dinv, *enc2_params(e1))
  z2y = _t2_call(1, emb3, z1, acc1y, dinv, *enc2_params(e2))
  acc2x = segsum_k(z2x, srcx, dstx).reshape(2, NPAD, D)
  acc2y = segsum_k(z2y, srcy, dsty).reshape(2, NPAD, D)
  lpx, gsx = _t3_call(0, acc2x, z2x, dinv, *enc3_params(e1), *mlp1p)
  lpy, gsy = _t3_call(1, acc2y, z2y, dinv, *enc3_params(e2), *mlp1p)

  mw = jnp.stack([m2['lin'][i]['W'] for i in range(3)])
  mb = jnp.stack([m2['lin'][i]['b'] for i in range(3)]).reshape(3, 1, D)
  mg = jnp.stack([m2['ln'][i]['g'] for i in range(3)]).reshape(3, 1, D)
  mbe = jnp.stack([m2['ln'][i]['b'] for i in range(3)]).reshape(3, 1, D)
  mal = jnp.stack([m2['prelu'][i] for i in range(3)]).reshape(3, 1, 1)
  msw = m2['short_W']
  msb = m2['short_b'].reshape(1, D)
  loss = _t4_call(lpx, lpy, gsx, gsy, mw, mb, mg, mbe, mal, msw, msb)
  return loss[0, 0]


# speedup vs baseline: 24.0803x; 1.0088x over previous
import functools

import numpy as np
import jax
import jax.numpy as jnp
from jax import lax
from jax.experimental import pallas as pl
from jax.experimental.pallas import tpu as pltpu
from jax.experimental.pallas import tpu_sc as plsc

D = 128
N = 10000
E = 320000
B = 16
NC = 2
NS = 16
W = 80
WE = 128
ERP = 2560
EPAD = ERP * WE - E
NWIN = ERP // NS
CH = 40
NPAD = 10240
LOG2 = float(np.log(2.0))


def _fill_zero_rows(zbuf):
  def row(r, _):
    for k in range(8):
      zbuf[r, pl.ds(k * 16, 16)] = jnp.zeros((16,), jnp.float32)
    return 0
  lax.fori_loop(0, 128, row, 0)


def _embdeg_body(table, ids2d, dst2d, embflat, deg, idsv, rows_v, dstv,
                 ones_v, zrow, deg_sh):
  c = lax.axis_index("c")
  s = lax.axis_index("s")

  def fill16(i, _):
    zrow[pl.ds(i * 16, 16)] = jnp.zeros((16,), jnp.float32)
    return 0
  lax.fori_loop(0, 40, fill16, 0)
  for i in range(8):
    ones_v[pl.ds(i * 16, 16)] = jnp.ones((16,), jnp.float32)

  pltpu.sync_copy(zrow, deg_sh.at[pl.ds(s * 640, 640)])
  plsc.subcore_barrier()

  pltpu.sync_copy(ids2d.at[pl.ds(c * 128 + s * 8, 8)], idsv)

  def gather_win(j, _):
    w = s * 8 + j
    @pl.when(w < 125)
    def _():
      pltpu.sync_copy(table.at[idsv.at[j]], rows_v)
      pltpu.sync_copy(rows_v, embflat.at[pl.ds(c * N + w * W, W)])
    return 0
  lax.fori_loop(0, 8, gather_win, 0)

  pltpu.sync_copy(dst2d.at[pl.ds(c * ERP + s * NWIN, NWIN)], dstv)

  def hist(j, _):
    pltpu.sync_copy(ones_v, deg_sh.at[dstv.at[j]], add=True)
    return 0
  lax.fori_loop(0, NWIN, hist, 0)

  plsc.subcore_barrier()
  pltpu.sync_copy(deg_sh.at[pl.ds(s * 640, 640)], deg.at[c, pl.ds(s * 640, 640)])


def _segsum_body(zflat, src2d, dst2d, accflat, srcv, dstv, buf0, buf1,
                 sg0, sg1, ss0, ss1, acc_sh):
  c = lax.axis_index("c")
  s = lax.axis_index("s")

  _fill_zero_rows(buf0)
  base = s * 640
  for q in range(5):
    pltpu.sync_copy(buf0, acc_sh.at[pl.ds(base + q * 128, 128)])
  plsc.subcore_barrier()

  erow = c * ERP + s * NWIN

  for chunk in range(NWIN // CH):
    pltpu.sync_copy(src2d.at[pl.ds(erow + chunk * CH, CH)], srcv)
    pltpu.sync_copy(dst2d.at[pl.ds(erow + chunk * CH, CH)], dstv)

    def g_desc(j, b, sem):
      return pltpu.make_async_copy(zflat.at[srcv.at[j]], b, sem)

    def s_desc(j, b, sem):
      return pltpu.make_async_copy(b, acc_sh.at[dstv.at[j]], sem)

    g_desc(0, buf0, sg0).start()

    def pair(k, _):
      w0 = 2 * k
      w1 = 2 * k + 1
      g_desc(w0, buf0, sg0).wait()

      @pl.when(k > 0)
      def _():
        s_desc(w1 - 2, buf1, ss1).wait()

      g_desc(w1, buf1, sg1).start()
      s_desc(w0, buf0, ss0).start(add=True)
      g_desc(w1, buf1, sg1).wait()
      s_desc(w0, buf0, ss0).wait()

      @pl.when(k < CH // 2 - 1)
      def _():
        g_desc(w0 + 2, buf0, sg0).start()

      s_desc(w1, buf1, ss1).start(add=True)
      return 0
    lax.fori_loop(0, CH // 2, pair, 0)
    s_desc(CH - 1, buf1, ss1).wait()

  plsc.subcore_barrier()
  pltpu.sync_copy(acc_sh.at[pl.ds(base, 640)],
                  accflat.at[pl.ds(c * NPAD + base, 640)])


@functools.lru_cache(maxsize=None)
def _sc_kernels():
  mesh = plsc.VectorSubcoreMesh(core_axis_name="c", subcore_axis_name="s",
                                num_cores=NC, num_subcores=NS)
  embdeg = functools.partial(
      pl.kernel,
      out_type=(
          jax.ShapeDtypeStruct((2 * N, D), jnp.float32),
          jax.ShapeDtypeStruct((2, NPAD), jnp.float32),
      ),
      mesh=mesh,
      scratch_types=(
          pltpu.VMEM((8, W), jnp.int32),
          pltpu.VMEM((W, D), jnp.float32),
          pltpu.VMEM((NWIN, WE), jnp.int32),
          pltpu.VMEM((WE,), jnp.float32),
          pltpu.VMEM((640,), jnp.float32),
          pltpu.VMEM_SHARED((NPAD,), jnp.float32),
      ),
  )(_embdeg_body)
  segsum = functools.partial(
      pl.kernel,
      out_type=jax.ShapeDtypeStruct((2 * NPAD, D), jnp.float32),
      mesh=mesh,
      scratch_types=(
          pltpu.VMEM((CH, WE), jnp.int32),
          pltpu.VMEM((CH, WE), jnp.int32),
          pltpu.VMEM((WE, D), jnp.float32),
          pltpu.VMEM((WE, D), jnp.float32),
          pltpu.SemaphoreType.DMA,
          pltpu.SemaphoreType.DMA,
          pltpu.SemaphoreType.DMA,
          pltpu.SemaphoreType.DMA,
          pltpu.VMEM_SHARED((NPAD, D), jnp.float32),
      ),
  )(_segsum_body)
  return embdeg, segsum


def _ln(x, g, b):
  m = jnp.mean(x, axis=-1, keepdims=True)
  v = jnp.mean((x - m) ** 2, axis=-1, keepdims=True)
  return (x - m) * lax.rsqrt(v + 1e-5) * g + b


def _matmul_t(x, w):
  return lax.dot_general(x, w, (((1,), (1,)), ((), ())),
                         preferred_element_type=jnp.float32)


def _softplus(x):
  return jnp.maximum(x, 0.0) + jnp.log1p(jnp.exp(-jnp.abs(x)))


def _t1_body(emb, w1, deg, z1, dinv):
  dv = lax.rsqrt(deg[0] + 1.0)
  z = _matmul_t(emb[0], w1[0]) * dv
  z1[...] = z[None]
  dinv[...] = dv[None]


def _t2_body(emb, z1, acc1, dinv, ws, bs, b1, g1, be1, w2, z2):
  dv = dinv[0]
  pre = dv * (acc1[0] + z1[0]) + b1[0]
  h = jax.nn.relu(_ln(pre, g1[0], be1[0]))
  u = _matmul_t(emb[0], ws[0]) + bs[0] + h
  z2[...] = (_matmul_t(u, w2[0]) * dv)[None]


def _t3_body(acc2, z2, dinv, b2, g2, be2, lw, lb, lg, lbe, lal, sw, sb,
             lp, gsum):
  dv = dinv[0]
  local = jax.nn.relu(_ln(dv * (acc2[0] + z2[0]) + b2[0], g2[0], be2[0]))
  rows = lax.broadcasted_iota(jnp.int32, (8, 5000), 1) // 625
  gid = lax.broadcasted_iota(jnp.int32, (8, 5000), 0)
  onehot = (rows == gid).astype(jnp.float32)
  gsum[...] = lax.dot_general(onehot, local, (((1,), (0,)), ((), ())),
                              preferred_element_type=jnp.float32)[None]
  h = local
  for i in range(3):
    a = lal[i]
    t = _ln(_matmul_t(h, lw[i]) + lb[i], lg[i], lbe[i])
    h = jnp.where(t >= 0, t, a * t)
  lp[...] = (h + _matmul_t(local, sw[...]) + sb[...])[None]


def _t4_body(lp, gsum, mw, mb, mg, mbe, mal, msw, msb, out):
  hblk = pl.program_id(0)

  def mlp2(xg):
    h = xg
    for i in range(3):
      t = _ln(_matmul_t(h, mw[i]) + mb[i], mg[i], mbe[i])
      h = jnp.where(t >= 0, t, mal[i] * t)
    return h + _matmul_t(xg, msw[...]) + msb[...]

  g1p = mlp2(gsum[0] / 625.0)
  g2p = mlp2(gsum[1] / 625.0)

  def contrib(l_blk, gp):
    res = _matmul_t(l_blk, gp)
    rowg = lax.broadcasted_iota(jnp.int32, (5000, B), 0) // 625 + hblk * 8
    colg = lax.broadcasted_iota(jnp.int32, (5000, B), 1)
    pos = (colg == rowg).astype(jnp.float32)
    spn = _softplus(-res)
    e_pos = jnp.sum(pos * (LOG2 - spn))
    e_neg = jnp.sum((1.0 - pos) * (spn + res - LOG2))
    return e_neg / (N * (B - 1)) - e_pos / N

  c = contrib(lp[0], g2p) + contrib(lp[1], g1p)

  cb = jnp.reshape(c, (1, 1))

  @pl.when(hblk == 0)
  def _():
    out[...] = cb

  @pl.when(hblk != 0)
  def _():
    out[...] = out[...] + cb


def _spec(shape, index_map):
  return pl.BlockSpec(shape, index_map)


def _t1_call(emb3, w1s, degv):
  return pl.pallas_call(
      _t1_body,
      grid=(2, 10),
      in_specs=[
          _spec((1, 1000, D), lambda v, i: (v, i, 0)),
          _spec((1, D, D), lambda v, i: (v, 0, 0)),
          _spec((1, 1000, 1), lambda v, i: (v, i, 0)),
      ],
      out_specs=[
          _spec((1, 1000, D), lambda v, i: (v, i, 0)),
          _spec((1, 1000, 1), lambda v, i: (v, i, 0)),
      ],
      out_shape=[
          jax.ShapeDtypeStruct((2, N, D), jnp.float32),
          jax.ShapeDtypeStruct((2, N, 1), jnp.float32),
      ],
  )(emb3, w1s, degv)


def _t2_call(emb3, z1, acc1, dinv, wss, bss, b1s, g1s, be1s, w2s):
  row = lambda v, i: (v, i, 0)
  par = lambda v, i: (v, 0, 0)
  return pl.pallas_call(
      _t2_body,
      grid=(2, 10),
      in_specs=[
          _spec((1, 1000, D), row), _spec((1, 1000, D), row),
          _spec((1, 1000, D), row), _spec((1, 1000, 1), row),
          _spec((1, D, D), par), _spec((1, 1, D), par), _spec((1, 1, D), par),
          _spec((1, 1, D), par), _spec((1, 1, D), par), _spec((1, D, D), par),
      ],
      out_specs=_spec((1, 1000, D), row),
      out_shape=jax.ShapeDtypeStruct((2, N, D), jnp.float32),
  )(emb3, z1, acc1, dinv, wss, bss, b1s, g1s, be1s, w2s)


def _t3_call(acc2, z2, dinv, b2s, g2s, be2s, lw, lb, lg, lbe, lal, sw, sb):
  row = lambda v, h: (v, h, 0)
  par = lambda v, h: (v, 0, 0)
  full3 = lambda v, h: (0, 0, 0)
  full2 = lambda v, h: (0, 0)
  return pl.pallas_call(
      _t3_body,
      grid=(2, 2),
      in_specs=[
          _spec((1, 5000, D), row), _spec((1, 5000, D), row),
          _spec((1, 5000, 1), row),
          _spec((1, 1, D), par), _spec((1, 1, D), par), _spec((1, 1, D), par),
          _spec((3, D, D), full3), _spec((3, 1, D), full3),
          _spec((3, 1, D), full3), _spec((3, 1, D), full3),
          _spec((3, 1, 1), full3),
          _spec((D, D), full2), _spec((1, D), full2),
      ],
      out_specs=[
          _spec((1, 5000, D), row),
          _spec((1, 8, D), row),
      ],
      out_shape=[
          jax.ShapeDtypeStruct((2, N, D), jnp.float32),
          jax.ShapeDtypeStruct((2, B, D), jnp.float32),
      ],
  )(acc2, z2, dinv, b2s, g2s, be2s, lw, lb, lg, lbe, lal, sw, sb)


def _t4_call(lp, gsum, mw, mb, mg, mbe, mal, msw, msb):
  full3 = lambda h: (0, 0, 0)
  full2 = lambda h: (0, 0)
  return pl.pallas_call(
      _t4_body,
      grid=(2,),
      in_specs=[
          _spec((2, 5000, D), lambda h: (0, h, 0)),
          _spec((2, B, D), full3),
          _spec((3, D, D), full3), _spec((3, 1, D), full3),
          _spec((3, 1, D), full3), _spec((3, 1, D), full3),
          _spec((3, 1, 1), full3),
          _spec((D, D), full2), _spec((1, D), full2),
      ],
      out_specs=_spec((1, 1), lambda h: (0, 0)),
      out_shape=jax.ShapeDtypeStruct((1, 1), jnp.float32),
  )(lp, gsum, mw, mb, mg, mbe, mal, msw, msb)


def _stack_enc(p, key, idx=None, sub=None):
  def leaf(enc):
    v = enc[key]
    if idx is not None:
      v = v[idx]
    if sub is not None:
      v = v[sub]
    return v
  return jnp.stack([leaf(p['enc1']), leaf(p['enc2'])])


def kernel(x, edge_index_x, ptr_x, y, edge_index_y, ptr_y, params):
  p = params
  table = p['emb']

  xi = x.astype(jnp.int32)
  yi = y.astype(jnp.int32)
  padi = jnp.zeros((NPAD - N,), jnp.int32)
  ids2d = jnp.concatenate([xi, padi, yi, padi]).reshape(2 * 128, W)

  sx, dx = edge_index_x[0], edge_index_x[1]
  sy, dy = edge_index_y[0], edge_index_y[1]
  ar = jnp.arange(EPAD, dtype=jnp.int32)
  pad_src = ar % 240
  pad_dst = N + ar % (NPAD - N)
  src2d = jnp.concatenate([sx, pad_src, sy + N, pad_src + N]).reshape(
      2 * ERP, WE)
  dst2d = jnp.concatenate([dx, pad_dst, dy, pad_dst]).reshape(2 * ERP, WE)

  embdeg_k, segsum_k = _sc_kernels()
  embflat, deg2 = embdeg_k(table, ids2d, dst2d)
  emb3 = embflat.reshape(2, N, D)
  degv = deg2.reshape(2, NPAD, 1)

  w1s = _stack_enc(p, 'conv', idx=0, sub='W')
  z1, dinv = _t1_call(emb3, w1s, degv)

  acc1 = segsum_k(z1.reshape(2 * N, D), src2d,
                  dst2d).reshape(2, NPAD, D)

  wss = _stack_enc(p, 'fc_skip_W')
  bss = _stack_enc(p, 'fc_skip_b').reshape(2, 1, D)
  b1s = _stack_enc(p, 'conv', idx=0, sub='b').reshape(2, 1, D)
  g1s = _stack_enc(p, 'ln', idx=0, sub='g').reshape(2, 1, D)
  be1s = _stack_enc(p, 'ln', idx=0, sub='b').reshape(2, 1, D)
  w2s = _stack_enc(p, 'conv', idx=1, sub='W')
  z2 = _t2_call(emb3, z1, acc1, dinv, wss, bss, b1s, g1s, be1s, w2s)

  acc2 = segsum_k(z2.reshape(2 * N, D), src2d,
                  dst2d).reshape(2, NPAD, D)

  b2s = _stack_enc(p, 'conv', idx=1, sub='b').reshape(2, 1, D)
  g2s = _stack_enc(p, 'ln', idx=1, sub='g').reshape(2, 1, D)
  be2s = _stack_enc(p, 'ln', idx=1, sub='b').reshape(2, 1, D)
  m1 = p['mlp1']
  lw = jnp.stack([m1['lin'][i]['W'] for i in range(3)])
  lb = jnp.stack([m1['lin'][i]['b'] for i in range(3)]).reshape(3, 1, D)
  lg = jnp.stack([m1['ln'][i]['g'] for i in range(3)]).reshape(3, 1, D)
  lbe = jnp.stack([m1['ln'][i]['b'] for i in range(3)]).reshape(3, 1, D)
  lal = jnp.stack([m1['prelu'][i] for i in range(3)]).reshape(3, 1, 1)
  sw = m1['short_W']
  sb = m1['short_b'].reshape(1, D)
  lp, gsum = _t3_call(acc2, z2, dinv, b2s, g2s, be2s, lw, lb, lg, lbe, lal,
                      sw, sb)

  m2 = p['mlp2']
  mw = jnp.stack([m2['lin'][i]['W'] for i in range(3)])
  mb = jnp.stack([m2['lin'][i]['b'] for i in range(3)]).reshape(3, 1, D)
  mg = jnp.stack([m2['ln'][i]['g'] for i in range(3)]).reshape(3, 1, D)
  mbe = jnp.stack([m2['ln'][i]['b'] for i in range(3)]).reshape(3, 1, D)
  mal = jnp.stack([m2['prelu'][i] for i in range(3)]).reshape(3, 1, 1)
  msw = m2['short_W']
  msb = m2['short_b'].reshape(1, D)
  loss = _t4_call(lp, gsum, mw, mb, mg, mbe, mal, msw, msb)
  return loss[0, 0]


# speedup vs baseline: 24.3349x; 1.0106x over previous
import functools

import numpy as np
import jax
import jax.numpy as jnp
from jax import lax
from jax.experimental import pallas as pl
from jax.experimental.pallas import tpu as pltpu
from jax.experimental.pallas import tpu_sc as plsc

D = 128
N = 10000
E = 320000
B = 16
NC = 2
NS = 16
W = 80
WE = 128
ERP = 2560
EPAD = ERP * WE - E
NWIN = ERP // NS
CH = 40
NPAD = 10240
LOG2 = float(np.log(2.0))


def _fill_zero_rows(zbuf):
  def row(r, _):
    for k in range(8):
      zbuf[r, pl.ds(k * 16, 16)] = jnp.zeros((16,), jnp.float32)
    return 0
  lax.fori_loop(0, 128, row, 0)


def _embdeg_body(table, ids2d, dst2d, embflat, deg, idsv, rows_v, dstv,
                 ones_v, zrow, hsem, deg_sh):
  c = lax.axis_index("c")
  s = lax.axis_index("s")

  def fill16(i, _):
    zrow[pl.ds(i * 16, 16)] = jnp.zeros((16,), jnp.float32)
    return 0
  lax.fori_loop(0, 40, fill16, 0)
  for i in range(8):
    ones_v[pl.ds(i * 16, 16)] = jnp.ones((16,), jnp.float32)

  pltpu.sync_copy(zrow, deg_sh.at[pl.ds(s * 640, 640)])
  plsc.subcore_barrier()

  pltpu.sync_copy(ids2d.at[pl.ds(c * 128 + s * 8, 8)], idsv)

  def gather_win(j, _):
    w = s * 8 + j
    @pl.when(w < 125)
    def _():
      pltpu.sync_copy(table.at[idsv.at[j]], rows_v)
      pltpu.sync_copy(rows_v, embflat.at[pl.ds(c * N + w * W, W)])
    return 0
  lax.fori_loop(0, 8, gather_win, 0)

  pltpu.sync_copy(dst2d.at[pl.ds(c * ERP + s * NWIN, NWIN)], dstv)

  def hist8(jc, _):
    for u in range(8):
      pltpu.make_async_copy(ones_v, deg_sh.at[dstv.at[jc * 8 + u]],
                            hsem).start(add=True)
    for u in range(8):
      pltpu.make_async_copy(ones_v, deg_sh.at[dstv.at[jc * 8 + u]],
                            hsem).wait()
    return 0
  lax.fori_loop(0, NWIN // 8, hist8, 0)

  plsc.subcore_barrier()
  pltpu.sync_copy(deg_sh.at[pl.ds(s * 640, 640)], deg.at[c, pl.ds(s * 640, 640)])


def _segsum_body(zflat, src2d, dst2d, accflat, srcv, dstv, buf0, buf1,
                 sg0, sg1, ss0, ss1, acc_sh):
  c = lax.axis_index("c")
  s = lax.axis_index("s")

  base = s * 640

  @pl.when(s < 15)
  def _():
    pltpu.sync_copy(zflat.at[pl.ds(c * N + base, 640)],
                    acc_sh.at[pl.ds(base, 640)])

  @pl.when(s == 15)
  def _():
    pltpu.sync_copy(zflat.at[pl.ds(c * N + 9600, 400)],
                    acc_sh.at[pl.ds(9600, 400)])
  plsc.subcore_barrier()

  erow = c * ERP + s * NWIN

  for chunk in range(NWIN // CH):
    pltpu.sync_copy(src2d.at[pl.ds(erow + chunk * CH, CH)], srcv)
    pltpu.sync_copy(dst2d.at[pl.ds(erow + chunk * CH, CH)], dstv)

    def g_desc(j, b, sem):
      return pltpu.make_async_copy(zflat.at[srcv.at[j]], b, sem)

    def s_desc(j, b, sem):
      return pltpu.make_async_copy(b, acc_sh.at[dstv.at[j]], sem)

    g_desc(0, buf0, sg0).start()

    def pair(k, _):
      w0 = 2 * k
      w1 = 2 * k + 1
      g_desc(w0, buf0, sg0).wait()

      @pl.when(k > 0)
      def _():
        s_desc(w1 - 2, buf1, ss1).wait()

      g_desc(w1, buf1, sg1).start()
      s_desc(w0, buf0, ss0).start(add=True)
      g_desc(w1, buf1, sg1).wait()
      s_desc(w0, buf0, ss0).wait()

      @pl.when(k < CH // 2 - 1)
      def _():
        g_desc(w0 + 2, buf0, sg0).start()

      s_desc(w1, buf1, ss1).start(add=True)
      return 0
    lax.fori_loop(0, CH // 2, pair, 0)
    s_desc(CH - 1, buf1, ss1).wait()

  plsc.subcore_barrier()

  @pl.when(s < 15)
  def _():
    pltpu.sync_copy(acc_sh.at[pl.ds(base, 640)],
                    accflat.at[pl.ds(c * NPAD + base, 640)])

  @pl.when(s == 15)
  def _():
    pltpu.sync_copy(acc_sh.at[pl.ds(9600, 400)],
                    accflat.at[pl.ds(c * NPAD + 9600, 400)])


@functools.lru_cache(maxsize=None)
def _sc_kernels():
  mesh = plsc.VectorSubcoreMesh(core_axis_name="c", subcore_axis_name="s",
                                num_cores=NC, num_subcores=NS)
  embdeg = functools.partial(
      pl.kernel,
      out_type=(
          jax.ShapeDtypeStruct((2 * N, D), jnp.float32),
          jax.ShapeDtypeStruct((2, NPAD), jnp.float32),
      ),
      mesh=mesh,
      scratch_types=(
          pltpu.VMEM((8, W), jnp.int32),
          pltpu.VMEM((W, D), jnp.float32),
          pltpu.VMEM((NWIN, WE), jnp.int32),
          pltpu.VMEM((WE,), jnp.float32),
          pltpu.VMEM((640,), jnp.float32),
          pltpu.SemaphoreType.DMA,
          pltpu.VMEM_SHARED((NPAD,), jnp.float32),
      ),
  )(_embdeg_body)
  segsum = functools.partial(
      pl.kernel,
      out_type=jax.ShapeDtypeStruct((2 * NPAD, D), jnp.float32),
      mesh=mesh,
      scratch_types=(
          pltpu.VMEM((CH, WE), jnp.int32),
          pltpu.VMEM((CH, WE), jnp.int32),
          pltpu.VMEM((WE, D), jnp.float32),
          pltpu.VMEM((WE, D), jnp.float32),
          pltpu.SemaphoreType.DMA,
          pltpu.SemaphoreType.DMA,
          pltpu.SemaphoreType.DMA,
          pltpu.SemaphoreType.DMA,
          pltpu.VMEM_SHARED((NPAD, D), jnp.float32),
      ),
  )(_segsum_body)
  return embdeg, segsum


def _ln(x, g, b):
  m = jnp.mean(x, axis=-1, keepdims=True)
  v = jnp.mean((x - m) ** 2, axis=-1, keepdims=True)
  return (x - m) * lax.rsqrt(v + 1e-5) * g + b


def _matmul_t(x, w):
  return lax.dot_general(x, w, (((1,), (1,)), ((), ())),
                         preferred_element_type=jnp.float32)


def _softplus(x):
  return jnp.maximum(x, 0.0) + jnp.log1p(jnp.exp(-jnp.abs(x)))


def _t1_body(emb, w1, deg, z1, dinv):
  dv = lax.rsqrt(deg[0] + 1.0)
  z = _matmul_t(emb[0], w1[0]) * dv
  z1[...] = z[None]
  dinv[...] = dv[None]


def _t2_body(emb, acc1, dinv, ws, bs, b1, g1, be1, w2, z2):
  dv = dinv[0]
  pre = dv * acc1[0] + b1[0]
  h = jax.nn.relu(_ln(pre, g1[0], be1[0]))
  u = _matmul_t(emb[0], ws[0]) + bs[0] + h
  z2[...] = (_matmul_t(u, w2[0]) * dv)[None]


def _t3_body(acc2, dinv, b2, g2, be2, lw, lb, lg, lbe, lal, sw, sb,
             lp, gsum):
  dv = dinv[0]
  local = jax.nn.relu(_ln(dv * acc2[0] + b2[0], g2[0], be2[0]))
  rows = lax.broadcasted_iota(jnp.int32, (8, 5000), 1) // 625
  gid = lax.broadcasted_iota(jnp.int32, (8, 5000), 0)
  onehot = (rows == gid).astype(jnp.float32)
  gsum[...] = lax.dot_general(onehot, local, (((1,), (0,)), ((), ())),
                              preferred_element_type=jnp.float32)[None]
  h = local
  for i in range(3):
    a = lal[i]
    t = _ln(_matmul_t(h, lw[i]) + lb[i], lg[i], lbe[i])
    h = jnp.where(t >= 0, t, a * t)
  lp[...] = (h + _matmul_t(local, sw[...]) + sb[...])[None]


def _t4_body(lp, gsum, mw, mb, mg, mbe, mal, msw, msb, out):
  hblk = pl.program_id(0)

  def mlp2(xg):
    h = xg
    for i in range(3):
      t = _ln(_matmul_t(h, mw[i]) + mb[i], mg[i], mbe[i])
      h = jnp.where(t >= 0, t, mal[i] * t)
    return h + _matmul_t(xg, msw[...]) + msb[...]

  g1p = mlp2(gsum[0] / 625.0)
  g2p = mlp2(gsum[1] / 625.0)

  def contrib(l_blk, gp):
    res = _matmul_t(l_blk, gp)
    rowg = lax.broadcasted_iota(jnp.int32, (5000, B), 0) // 625 + hblk * 8
    colg = lax.broadcasted_iota(jnp.int32, (5000, B), 1)
    pos = (colg == rowg).astype(jnp.float32)
    spn = _softplus(-res)
    e_pos = jnp.sum(pos * (LOG2 - spn))
    e_neg = jnp.sum((1.0 - pos) * (spn + res - LOG2))
    return e_neg / (N * (B - 1)) - e_pos / N

  c = contrib(lp[0], g2p) + contrib(lp[1], g1p)

  cb = jnp.reshape(c, (1, 1))

  @pl.when(hblk == 0)
  def _():
    out[...] = cb

  @pl.when(hblk != 0)
  def _():
    out[...] = out[...] + cb


def _spec(shape, index_map):
  return pl.BlockSpec(shape, index_map)


def _t1_call(emb3, w1s, degv):
  return pl.pallas_call(
      _t1_body,
      grid=(2, 10),
      in_specs=[
          _spec((1, 1000, D), lambda v, i: (v, i, 0)),
          _spec((1, D, D), lambda v, i: (v, 0, 0)),
          _spec((1, 1000, 1), lambda v, i: (v, i, 0)),
      ],
      out_specs=[
          _spec((1, 1000, D), lambda v, i: (v, i, 0)),
          _spec((1, 1000, 1), lambda v, i: (v, i, 0)),
      ],
      out_shape=[
          jax.ShapeDtypeStruct((2, N, D), jnp.float32),
          jax.ShapeDtypeStruct((2, N, 1), jnp.float32),
      ],
  )(emb3, w1s, degv)


def _t2_call(emb3, acc1, dinv, wss, bss, b1s, g1s, be1s, w2s):
  row = lambda v, i: (v, i, 0)
  par = lambda v, i: (v, 0, 0)
  return pl.pallas_call(
      _t2_body,
      grid=(2, 10),
      in_specs=[
          _spec((1, 1000, D), row), _spec((1, 1000, D), row),
          _spec((1, 1000, 1), row),
          _spec((1, D, D), par), _spec((1, 1, D), par), _spec((1, 1, D), par),
          _spec((1, 1, D), par), _spec((1, 1, D), par), _spec((1, D, D), par),
      ],
      out_specs=_spec((1, 1000, D), row),
      out_shape=jax.ShapeDtypeStruct((2, N, D), jnp.float32),
  )(emb3, acc1, dinv, wss, bss, b1s, g1s, be1s, w2s)


def _t3_call(acc2, dinv, b2s, g2s, be2s, lw, lb, lg, lbe, lal, sw, sb):
  row = lambda v, h: (v, h, 0)
  par = lambda v, h: (v, 0, 0)
  full3 = lambda v, h: (0, 0, 0)
  full2 = lambda v, h: (0, 0)
  return pl.pallas_call(
      _t3_body,
      grid=(2, 2),
      in_specs=[
          _spec((1, 5000, D), row),
          _spec((1, 5000, 1), row),
          _spec((1, 1, D), par), _spec((1, 1, D), par), _spec((1, 1, D), par),
          _spec((3, D, D), full3), _spec((3, 1, D), full3),
          _spec((3, 1, D), full3), _spec((3, 1, D), full3),
          _spec((3, 1, 1), full3),
          _spec((D, D), full2), _spec((1, D), full2),
      ],
      out_specs=[
          _spec((1, 5000, D), row),
          _spec((1, 8, D), row),
      ],
      out_shape=[
          jax.ShapeDtypeStruct((2, N, D), jnp.float32),
          jax.ShapeDtypeStruct((2, B, D), jnp.float32),
      ],
  )(acc2, dinv, b2s, g2s, be2s, lw, lb, lg, lbe, lal, sw, sb)


def _t4_call(lp, gsum, mw, mb, mg, mbe, mal, msw, msb):
  full3 = lambda h: (0, 0, 0)
  full2 = lambda h: (0, 0)
  return pl.pallas_call(
      _t4_body,
      grid=(2,),
      in_specs=[
          _spec((2, 5000, D), lambda h: (0, h, 0)),
          _spec((2, B, D), full3),
          _spec((3, D, D), full3), _spec((3, 1, D), full3),
          _spec((3, 1, D), full3), _spec((3, 1, D), full3),
          _spec((3, 1, 1), full3),
          _spec((D, D), full2), _spec((1, D), full2),
      ],
      out_specs=_spec((1, 1), lambda h: (0, 0)),
      out_shape=jax.ShapeDtypeStruct((1, 1), jnp.float32),
  )(lp, gsum, mw, mb, mg, mbe, mal, msw, msb)


def _stack_enc(p, key, idx=None, sub=None):
  def leaf(enc):
    v = enc[key]
    if idx is not None:
      v = v[idx]
    if sub is not None:
      v = v[sub]
    return v
  return jnp.stack([leaf(p['enc1']), leaf(p['enc2'])])


def kernel(x, edge_index_x, ptr_x, y, edge_index_y, ptr_y, params):
  p = params
  table = p['emb']

  xi = x.astype(jnp.int32)
  yi = y.astype(jnp.int32)
  padi = jnp.zeros((NPAD - N,), jnp.int32)
  ids2d = jnp.concatenate([xi, padi, yi, padi]).reshape(2 * 128, W)

  sx, dx = edge_index_x[0], edge_index_x[1]
  sy, dy = edge_index_y[0], edge_index_y[1]
  ar = jnp.arange(EPAD, dtype=jnp.int32)
  pad_src = ar % 240
  pad_dst = N + ar % (NPAD - N)
  src2d = jnp.concatenate([sx, pad_src, sy + N, pad_src + N]).reshape(
      2 * ERP, WE)
  dst2d = jnp.concatenate([dx, pad_dst, dy, pad_dst]).reshape(2 * ERP, WE)

  embdeg_k, segsum_k = _sc_kernels()
  embflat, deg2 = embdeg_k(table, ids2d, dst2d)
  emb3 = embflat.reshape(2, N, D)
  degv = deg2.reshape(2, NPAD, 1)

  w1s = _stack_enc(p, 'conv', idx=0, sub='W')
  z1, dinv = _t1_call(emb3, w1s, degv)

  acc1 = segsum_k(z1.reshape(2 * N, D), src2d,
                  dst2d).reshape(2, NPAD, D)

  wss = _stack_enc(p, 'fc_skip_W')
  bss = _stack_enc(p, 'fc_skip_b').reshape(2, 1, D)
  b1s = _stack_enc(p, 'conv', idx=0, sub='b').reshape(2, 1, D)
  g1s = _stack_enc(p, 'ln', idx=0, sub='g').reshape(2, 1, D)
  be1s = _stack_enc(p, 'ln', idx=0, sub='b').reshape(2, 1, D)
  w2s = _stack_enc(p, 'conv', idx=1, sub='W')
  z2 = _t2_call(emb3, acc1, dinv, wss, bss, b1s, g1s, be1s, w2s)

  acc2 = segsum_k(z2.reshape(2 * N, D), src2d,
                  dst2d).reshape(2, NPAD, D)

  b2s = _stack_enc(p, 'conv', idx=1, sub='b').reshape(2, 1, D)
  g2s = _stack_enc(p, 'ln', idx=1, sub='g').reshape(2, 1, D)
  be2s = _stack_enc(p, 'ln', idx=1, sub='b').reshape(2, 1, D)
  m1 = p['mlp1']
  lw = jnp.stack([m1['lin'][i]['W'] for i in range(3)])
  lb = jnp.stack([m1['lin'][i]['b'] for i in range(3)]).reshape(3, 1, D)
  lg = jnp.stack([m1['ln'][i]['g'] for i in range(3)]).reshape(3, 1, D)
  lbe = jnp.stack([m1['ln'][i]['b'] for i in range(3)]).reshape(3, 1, D)
  lal = jnp.stack([m1['prelu'][i] for i in range(3)]).reshape(3, 1, 1)
  sw = m1['short_W']
  sb = m1['short_b'].reshape(1, D)
  lp, gsum = _t3_call(acc2, dinv, b2s, g2s, be2s, lw, lb, lg, lbe, lal,
                      sw, sb)

  m2 = p['mlp2']
  mw = jnp.stack([m2['lin'][i]['W'] for i in range(3)])
  mb = jnp.stack([m2['lin'][i]['b'] for i in range(3)]).reshape(3, 1, D)
  mg = jnp.stack([m2['ln'][i]['g'] for i in range(3)]).reshape(3, 1, D)
  mbe = jnp.stack([m2['ln'][i]['b'] for i in range(3)]).reshape(3, 1, D)
  mal = jnp.stack([m2['prelu'][i] for i in range(3)]).reshape(3, 1, 1)
  msw = m2['short_W']
  msb = m2['short_b'].reshape(1, D)
  loss = _t4_call(lp, gsum, mw, mb, mg, mbe, mal, msw, msb)
  return loss[0, 0]


# speedup vs baseline: 24.3956x; 1.0025x over previous
import functools

import numpy as np
import jax
import jax.numpy as jnp
from jax import lax
from jax.experimental import pallas as pl
from jax.experimental.pallas import tpu as pltpu
from jax.experimental.pallas import tpu_sc as plsc

D = 128
N = 10000
E = 320000
B = 16
NC = 2
NS = 16
W = 80
WE = 128
ERP = 2560
EPAD = ERP * WE - E
NWIN = ERP // NS
CH = 40
NPAD = 10240
LOG2 = float(np.log(2.0))


def _fill_zero_rows(zbuf):
  def row(r, _):
    for k in range(8):
      zbuf[r, pl.ds(k * 16, 16)] = jnp.zeros((16,), jnp.float32)
    return 0
  lax.fori_loop(0, 128, row, 0)


def _embdeg_body(table, ids2d, dst2d, embflat, deg, idsv, rows_v, dstv,
                 ones_v, zrow, hsem, deg_sh):
  c = lax.axis_index("c")
  s = lax.axis_index("s")

  def fill16(i, _):
    zrow[pl.ds(i * 16, 16)] = jnp.zeros((16,), jnp.float32)
    return 0
  lax.fori_loop(0, 40, fill16, 0)
  for i in range(8):
    ones_v[pl.ds(i * 16, 16)] = jnp.ones((16,), jnp.float32)

  pltpu.sync_copy(zrow, deg_sh.at[pl.ds(s * 640, 640)])
  plsc.subcore_barrier()

  pltpu.sync_copy(ids2d.at[pl.ds(c * 128 + s * 8, 8)], idsv)

  def gather_win(j, _):
    w = s * 8 + j
    @pl.when(w < 125)
    def _():
      pltpu.sync_copy(table.at[idsv.at[j]], rows_v)
      pltpu.sync_copy(rows_v, embflat.at[pl.ds(c * N + w * W, W)])
    return 0
  lax.fori_loop(0, 8, gather_win, 0)

  pltpu.sync_copy(dst2d.at[pl.ds(c * ERP + s * NWIN, NWIN)], dstv)

  def hist8(jc, _):
    for u in range(8):
      pltpu.make_async_copy(ones_v, deg_sh.at[dstv.at[jc * 8 + u]],
                            hsem).start(add=True)
    for u in range(8):
      pltpu.make_async_copy(ones_v, deg_sh.at[dstv.at[jc * 8 + u]],
                            hsem).wait()
    return 0
  lax.fori_loop(0, NWIN // 8, hist8, 0)

  plsc.subcore_barrier()
  pltpu.sync_copy(deg_sh.at[pl.ds(s * 640, 640)], deg.at[c, pl.ds(s * 640, 640)])


def _segsum_body(zflat, src2d, dst2d, accflat, srcv, dstv, buf0, buf1,
                 sg0, sg1, ss0, ss1, acc_sh):
  c = lax.axis_index("c")
  s = lax.axis_index("s")

  base = s * 640

  @pl.when(s < 15)
  def _():
    pltpu.sync_copy(zflat.at[pl.ds(c * N + base, 640)],
                    acc_sh.at[pl.ds(base, 640)])

  @pl.when(s == 15)
  def _():
    pltpu.sync_copy(zflat.at[pl.ds(c * N + 9600, 400)],
                    acc_sh.at[pl.ds(9600, 400)])
  plsc.subcore_barrier()

  erow = c * ERP + s * NWIN

  for chunk in range(NWIN // CH):
    pltpu.sync_copy(src2d.at[pl.ds(erow + chunk * CH, CH)], srcv)
    pltpu.sync_copy(dst2d.at[pl.ds(erow + chunk * CH, CH)], dstv)

    def g_desc(j, b, sem):
      return pltpu.make_async_copy(zflat.at[srcv.at[j]], b, sem)

    def s_desc(j, b, sem):
      return pltpu.make_async_copy(b, acc_sh.at[dstv.at[j]], sem)

    g_desc(0, buf0, sg0).start()

    def pair(k, _):
      w0 = 2 * k
      w1 = 2 * k + 1
      g_desc(w0, buf0, sg0).wait()
      s_desc(w0, buf0, ss0).start(add=True)

      @pl.when(k > 0)
      def _():
        s_desc(w1 - 2, buf1, ss1).wait()

      g_desc(w1, buf1, sg1).start()
      g_desc(w1, buf1, sg1).wait()
      s_desc(w1, buf1, ss1).start(add=True)
      s_desc(w0, buf0, ss0).wait()

      @pl.when(k < CH // 2 - 1)
      def _():
        g_desc(w0 + 2, buf0, sg0).start()

      return 0
    lax.fori_loop(0, CH // 2, pair, 0)
    s_desc(CH - 1, buf1, ss1).wait()

  plsc.subcore_barrier()

  @pl.when(s < 15)
  def _():
    pltpu.sync_copy(acc_sh.at[pl.ds(base, 640)],
                    accflat.at[pl.ds(c * NPAD + base, 640)])

  @pl.when(s == 15)
  def _():
    pltpu.sync_copy(acc_sh.at[pl.ds(9600, 400)],
                    accflat.at[pl.ds(c * NPAD + 9600, 400)])


@functools.lru_cache(maxsize=None)
def _sc_kernels():
  mesh = plsc.VectorSubcoreMesh(core_axis_name="c", subcore_axis_name="s",
                                num_cores=NC, num_subcores=NS)
  embdeg = functools.partial(
      pl.kernel,
      out_type=(
          jax.ShapeDtypeStruct((2 * N, D), jnp.float32),
          jax.ShapeDtypeStruct((2, NPAD), jnp.float32),
      ),
      mesh=mesh,
      scratch_types=(
          pltpu.VMEM((8, W), jnp.int32),
          pltpu.VMEM((W, D), jnp.float32),
          pltpu.VMEM((NWIN, WE), jnp.int32),
          pltpu.VMEM((WE,), jnp.float32),
          pltpu.VMEM((640,), jnp.float32),
          pltpu.SemaphoreType.DMA,
          pltpu.VMEM_SHARED((NPAD,), jnp.float32),
      ),
  )(_embdeg_body)
  segsum = functools.partial(
      pl.kernel,
      out_type=jax.ShapeDtypeStruct((2 * NPAD, D), jnp.float32),
      mesh=mesh,
      scratch_types=(
          pltpu.VMEM((CH, WE), jnp.int32),
          pltpu.VMEM((CH, WE), jnp.int32),
          pltpu.VMEM((WE, D), jnp.float32),
          pltpu.VMEM((WE, D), jnp.float32),
          pltpu.SemaphoreType.DMA,
          pltpu.SemaphoreType.DMA,
          pltpu.SemaphoreType.DMA,
          pltpu.SemaphoreType.DMA,
          pltpu.VMEM_SHARED((NPAD, D), jnp.float32),
      ),
  )(_segsum_body)
  return embdeg, segsum


def _ln(x, g, b):
  m = jnp.mean(x, axis=-1, keepdims=True)
  v = jnp.mean((x - m) ** 2, axis=-1, keepdims=True)
  return (x - m) * lax.rsqrt(v + 1e-5) * g + b


def _matmul_t(x, w):
  return lax.dot_general(x, w, (((1,), (1,)), ((), ())),
                         preferred_element_type=jnp.float32)


def _softplus(x):
  return jnp.maximum(x, 0.0) + jnp.log1p(jnp.exp(-jnp.abs(x)))


def _t1_body(emb, w1, deg, z1, dinv):
  dv = lax.rsqrt(deg[0] + 1.0)
  z = _matmul_t(emb[0], w1[0]) * dv
  z1[...] = z[None]
  dinv[...] = dv[None]


def _t2_body(emb, acc1, dinv, ws, bs, b1, g1, be1, w2, z2):
  dv = dinv[0]
  pre = dv * acc1[0] + b1[0]
  h = jax.nn.relu(_ln(pre, g1[0], be1[0]))
  u = _matmul_t(emb[0], ws[0]) + bs[0] + h
  z2[...] = (_matmul_t(u, w2[0]) * dv)[None]


def _t3_body(acc2, dinv, b2, g2, be2, lw, lb, lg, lbe, lal, sw, sb,
             lp, gsum):
  dv = dinv[0]
  local = jax.nn.relu(_ln(dv * acc2[0] + b2[0], g2[0], be2[0]))
  rows = lax.broadcasted_iota(jnp.int32, (8, 5000), 1) // 625
  gid = lax.broadcasted_iota(jnp.int32, (8, 5000), 0)
  onehot = (rows == gid).astype(jnp.float32)
  gsum[...] = lax.dot_general(onehot, local, (((1,), (0,)), ((), ())),
                              preferred_element_type=jnp.float32)[None]
  h = local
  for i in range(3):
    a = lal[i]
    t = _ln(_matmul_t(h, lw[i]) + lb[i], lg[i], lbe[i])
    h = jnp.where(t >= 0, t, a * t)
  lp[...] = (h + _matmul_t(local, sw[...]) + sb[...])[None]


def _t4_body(lp, gsum, mw, mb, mg, mbe, mal, msw, msb, out):
  hblk = pl.program_id(0)

  def mlp2(xg):
    h = xg
    for i in range(3):
      t = _ln(_matmul_t(h, mw[i]) + mb[i], mg[i], mbe[i])
      h = jnp.where(t >= 0, t, mal[i] * t)
    return h + _matmul_t(xg, msw[...]) + msb[...]

  g1p = mlp2(gsum[0] / 625.0)
  g2p = mlp2(gsum[1] / 625.0)

  def contrib(l_blk, gp):
    res = _matmul_t(l_blk, gp)
    rowg = lax.broadcasted_iota(jnp.int32, (5000, B), 0) // 625 + hblk * 8
    colg = lax.broadcasted_iota(jnp.int32, (5000, B), 1)
    pos = (colg == rowg).astype(jnp.float32)
    spn = _softplus(-res)
    e_pos = jnp.sum(pos * (LOG2 - spn))
    e_neg = jnp.sum((1.0 - pos) * (spn + res - LOG2))
    return e_neg / (N * (B - 1)) - e_pos / N

  c = contrib(lp[0], g2p) + contrib(lp[1], g1p)

  cb = jnp.reshape(c, (1, 1))

  @pl.when(hblk == 0)
  def _():
    out[...] = cb

  @pl.when(hblk != 0)
  def _():
    out[...] = out[...] + cb


def _spec(shape, index_map):
  return pl.BlockSpec(shape, index_map)


def _t1_call(emb3, w1s, degv):
  return pl.pallas_call(
      _t1_body,
      grid=(2, 10),
      in_specs=[
          _spec((1, 1000, D), lambda v, i: (v, i, 0)),
          _spec((1, D, D), lambda v, i: (v, 0, 0)),
          _spec((1, 1000, 1), lambda v, i: (v, i, 0)),
      ],
      out_specs=[
          _spec((1, 1000, D), lambda v, i: (v, i, 0)),
          _spec((1, 1000, 1), lambda v, i: (v, i, 0)),
      ],
      out_shape=[
          jax.ShapeDtypeStruct((2, N, D), jnp.float32),
          jax.ShapeDtypeStruct((2, N, 1), jnp.float32),
      ],
  )(emb3, w1s, degv)


def _t2_call(emb3, acc1, dinv, wss, bss, b1s, g1s, be1s, w2s):
  row = lambda v, i: (v, i, 0)
  par = lambda v, i: (v, 0, 0)
  return pl.pallas_call(
      _t2_body,
      grid=(2, 10),
      in_specs=[
          _spec((1, 1000, D), row), _spec((1, 1000, D), row),
          _spec((1, 1000, 1), row),
          _spec((1, D, D), par), _spec((1, 1, D), par), _spec((1, 1, D), par),
          _spec((1, 1, D), par), _spec((1, 1, D), par), _spec((1, D, D), par),
      ],
      out_specs=_spec((1, 1000, D), row),
      out_shape=jax.ShapeDtypeStruct((2, N, D), jnp.float32),
  )(emb3, acc1, dinv, wss, bss, b1s, g1s, be1s, w2s)


def _t3_call(acc2, dinv, b2s, g2s, be2s, lw, lb, lg, lbe, lal, sw, sb):
  row = lambda v, h: (v, h, 0)
  par = lambda v, h: (v, 0, 0)
  full3 = lambda v, h: (0, 0, 0)
  full2 = lambda v, h: (0, 0)
  return pl.pallas_call(
      _t3_body,
      grid=(2, 2),
      in_specs=[
          _spec((1, 5000, D), row),
          _spec((1, 5000, 1), row),
          _spec((1, 1, D), par), _spec((1, 1, D), par), _spec((1, 1, D), par),
          _spec((3, D, D), full3), _spec((3, 1, D), full3),
          _spec((3, 1, D), full3), _spec((3, 1, D), full3),
          _spec((3, 1, 1), full3),
          _spec((D, D), full2), _spec((1, D), full2),
      ],
      out_specs=[
          _spec((1, 5000, D), row),
          _spec((1, 8, D), row),
      ],
      out_shape=[
          jax.ShapeDtypeStruct((2, N, D), jnp.float32),
          jax.ShapeDtypeStruct((2, B, D), jnp.float32),
      ],
  )(acc2, dinv, b2s, g2s, be2s, lw, lb, lg, lbe, lal, sw, sb)


def _t4_call(lp, gsum, mw, mb, mg, mbe, mal, msw, msb):
  full3 = lambda h: (0, 0, 0)
  full2 = lambda h: (0, 0)
  return pl.pallas_call(
      _t4_body,
      grid=(2,),
      in_specs=[
          _spec((2, 5000, D), lambda h: (0, h, 0)),
          _spec((2, B, D), full3),
          _spec((3, D, D), full3), _spec((3, 1, D), full3),
          _spec((3, 1, D), full3), _spec((3, 1, D), full3),
          _spec((3, 1, 1), full3),
          _spec((D, D), full2), _spec((1, D), full2),
      ],
      out_specs=_spec((1, 1), lambda h: (0, 0)),
      out_shape=jax.ShapeDtypeStruct((1, 1), jnp.float32),
  )(lp, gsum, mw, mb, mg, mbe, mal, msw, msb)


def _stack_enc(p, key, idx=None, sub=None):
  def leaf(enc):
    v = enc[key]
    if idx is not None:
      v = v[idx]
    if sub is not None:
      v = v[sub]
    return v
  return jnp.stack([leaf(p['enc1']), leaf(p['enc2'])])


def kernel(x, edge_index_x, ptr_x, y, edge_index_y, ptr_y, params):
  p = params
  table = p['emb']

  xi = x.astype(jnp.int32)
  yi = y.astype(jnp.int32)
  padi = jnp.zeros((NPAD - N,), jnp.int32)
  ids2d = jnp.concatenate([xi, padi, yi, padi]).reshape(2 * 128, W)

  sx, dx = edge_index_x[0], edge_index_x[1]
  sy, dy = edge_index_y[0], edge_index_y[1]
  ar = jnp.arange(EPAD, dtype=jnp.int32)
  pad_src = ar % 240
  pad_dst = N + ar % (NPAD - N)
  src2d = jnp.concatenate([sx, pad_src, sy + N, pad_src + N]).reshape(
      2 * ERP, WE)
  dst2d = jnp.concatenate([dx, pad_dst, dy, pad_dst]).reshape(2 * ERP, WE)

  embdeg_k, segsum_k = _sc_kernels()
  embflat, deg2 = embdeg_k(table, ids2d, dst2d)
  emb3 = embflat.reshape(2, N, D)
  degv = deg2.reshape(2, NPAD, 1)

  w1s = _stack_enc(p, 'conv', idx=0, sub='W')
  z1, dinv = _t1_call(emb3, w1s, degv)

  acc1 = segsum_k(z1.reshape(2 * N, D), src2d,
                  dst2d).reshape(2, NPAD, D)

  wss = _stack_enc(p, 'fc_skip_W')
  bss = _stack_enc(p, 'fc_skip_b').reshape(2, 1, D)
  b1s = _stack_enc(p, 'conv', idx=0, sub='b').reshape(2, 1, D)
  g1s = _stack_enc(p, 'ln', idx=0, sub='g').reshape(2, 1, D)
  be1s = _stack_enc(p, 'ln', idx=0, sub='b').reshape(2, 1, D)
  w2s = _stack_enc(p, 'conv', idx=1, sub='W')
  z2 = _t2_call(emb3, acc1, dinv, wss, bss, b1s, g1s, be1s, w2s)

  acc2 = segsum_k(z2.reshape(2 * N, D), src2d,
                  dst2d).reshape(2, NPAD, D)

  b2s = _stack_enc(p, 'conv', idx=1, sub='b').reshape(2, 1, D)
  g2s = _stack_enc(p, 'ln', idx=1, sub='g').reshape(2, 1, D)
  be2s = _stack_enc(p, 'ln', idx=1, sub='b').reshape(2, 1, D)
  m1 = p['mlp1']
  lw = jnp.stack([m1['lin'][i]['W'] for i in range(3)])
  lb = jnp.stack([m1['lin'][i]['b'] for i in range(3)]).reshape(3, 1, D)
  lg = jnp.stack([m1['ln'][i]['g'] for i in range(3)]).reshape(3, 1, D)
  lbe = jnp.stack([m1['ln'][i]['b'] for i in range(3)]).reshape(3, 1, D)
  lal = jnp.stack([m1['prelu'][i] for i in range(3)]).reshape(3, 1, 1)
  sw = m1['short_W']
  sb = m1['short_b'].reshape(1, D)
  lp, gsum = _t3_call(acc2, dinv, b2s, g2s, be2s, lw, lb, lg, lbe, lal,
                      sw, sb)

  m2 = p['mlp2']
  mw = jnp.stack([m2['lin'][i]['W'] for i in range(3)])
  mb = jnp.stack([m2['lin'][i]['b'] for i in range(3)]).reshape(3, 1, D)
  mg = jnp.stack([m2['ln'][i]['g'] for i in range(3)]).reshape(3, 1, D)
  mbe = jnp.stack([m2['ln'][i]['b'] for i in range(3)]).reshape(3, 1, D)
  mal = jnp.stack([m2['prelu'][i] for i in range(3)]).reshape(3, 1, 1)
  msw = m2['short_W']
  msb = m2['short_b'].reshape(1, D)
  loss = _t4_call(lp, gsum, mw, mb, mg, mbe, mal, msw, msb)
  return loss[0, 0]
